# Initial kernel scaffold; baseline (speedup 1.0000x reference)
import jax
import jax.numpy as jnp
from jax.experimental import pallas as pl


def kernel(x, params, edge_index, cluster_indices, cluster_mask):
    raise NotImplementedError("write your pallas kernel here")


# speedup vs baseline: 7.4097x; 7.4097x over previous
import dataclasses
import functools

import jax
import jax.numpy as jnp
from jax import lax
from jax.experimental import pallas as pl
from jax.experimental.pallas import tpu as pltpu
from jax.experimental.pallas import tpu_sc as plsc

N = 10000
E = 160000
D = 128
K = 32
C = 313
HEADS = 8
DEPTH = 8
NUM_CLASSES = 10

NPAD = 10240
EPAD = 180224
EW32 = EPAD // 32
EW16 = EPAD // 16
NROWS = NPAD // 128
DUMMY = N + 64
SPAD = 320
F32 = jnp.float32

_mesh = plsc.VectorSubcoreMesh(core_axis_name="c", subcore_axis_name="s")

_sc_params = pltpu.CompilerParams(use_tc_tiling_on_sc=False)
if "needs_layout_passes" in pltpu.CompilerParams.__dataclass_fields__:
    _sc_params = dataclasses.replace(_sc_params, needs_layout_passes=False)


def _edge_body(z_hbm, s_hbm, d_hbm, src2_hbm, dst2_hbm,
               iota_hbm, out_hbm,
               s_loc, d_loc, srcl_loc, dstl_loc,
               den_loc, alpha_loc, rowbuf, mx_loc, mred_loc, iota_loc,
               out_sh, den_sh, mx_sh):
    cid = lax.axis_index("c")
    tid = lax.axis_index("s")

    pltpu.sync_copy(s_hbm, s_loc)
    pltpu.sync_copy(d_hbm, d_loc)
    pltpu.sync_copy(src2_hbm.at[2 * tid], srcl_loc.at[pl.ds(0, 44)])
    pltpu.sync_copy(src2_hbm.at[2 * tid + 1], srcl_loc.at[pl.ds(44, 44)])
    pltpu.sync_copy(dst2_hbm.at[2 * tid], dstl_loc.at[pl.ds(0, 44)])
    pltpu.sync_copy(dst2_hbm.at[2 * tid + 1], dstl_loc.at[pl.ds(44, 44)])
    pltpu.sync_copy(iota_hbm, iota_loc)

    zero16 = jnp.zeros((16,), F32)

    @pl.loop(0, NROWS)
    def _(r):
        for cc in range(8):
            den_loc[r, pl.ds(cc * 16, 16)] = zero16

    @pl.when(tid == 0)
    def _():
        pltpu.sync_copy(den_loc, den_sh)

    mx_loc[...] = jnp.full((16,), -3e38, F32)

    @pl.loop(0, 88)
    def _(r):
        for cc in range(8):
            si = srcl_loc[r, pl.ds(cc * 16, 16)]
            di = dstl_loc[r, pl.ds(cc * 16, 16)]
            ev = plsc.load_gather(s_loc, [si]) + plsc.load_gather(d_loc, [di])
            ev = jnp.where(ev > 0, ev, 0.2 * ev)
            mx_loc[...] = jnp.maximum(mx_loc[...], ev)

    pltpu.sync_copy(mx_loc, mx_sh.at[tid])
    plsc.subcore_barrier()
    pltpu.sync_copy(mx_sh, mred_loc)
    mv = mred_loc[0, :]
    for t in range(1, 16):
        mv = jnp.maximum(mv, mred_loc[t, :])
    mscal = jnp.max(mv)
    mvec = jnp.full((16,), mscal, F32)

    @pl.loop(0, 88)
    def _(r):
        for cc in range(8):
            si = srcl_loc[r, pl.ds(cc * 16, 16)]
            di = dstl_loc[r, pl.ds(cc * 16, 16)]
            ev = plsc.load_gather(s_loc, [si]) + plsc.load_gather(d_loc, [di])
            ev = jnp.where(ev > 0, ev, 0.2 * ev)
            ex = jnp.exp(ev - mvec)
            plsc.addupdate_scatter(den_loc, [di >> 7, di & 127], ex)

    pltpu.sync_copy(den_loc, den_sh.at[iota_loc.at[0]], add=True)
    plsc.subcore_barrier()
    pltpu.sync_copy(den_sh, den_loc)

    @pl.loop(0, 44)
    def _(r):
        for cc in range(8):
            si = srcl_loc[cid * 44 + r, pl.ds(cc * 16, 16)]
            di = dstl_loc[cid * 44 + r, pl.ds(cc * 16, 16)]
            ev = plsc.load_gather(s_loc, [si]) + plsc.load_gather(d_loc, [di])
            ev = jnp.where(ev > 0, ev, 0.2 * ev)
            ex = jnp.exp(ev - mvec)
            dn = plsc.load_gather(den_loc, [di >> 7, di & 127])
            alpha_loc[pl.ds(r * 128 + cc * 16, 16)] = ex / (dn + 1e-16)

    for h in range(2):
        @pl.loop(0, 128)
        def _(r):
            for cc in range(4):
                rowbuf[r, pl.ds(cc * 16, 16)] = zero16
        for c5 in range(5):
            pltpu.sync_copy(rowbuf, out_sh.at[pl.ds(tid * 640 + c5 * 128, 128)])
        plsc.subcore_barrier()

        @pl.loop(0, 44)
        def _(j):
            pltpu.sync_copy(z_hbm.at[h].at[srcl_loc.at[cid * 44 + j]], rowbuf)

            @pl.loop(0, 128)
            def _(r):
                av = plsc.load_gather(alpha_loc, [jnp.full((16,), j * 128 + r,
                                                           jnp.int32)])
                for cc in range(4):
                    rowbuf[r, pl.ds(cc * 16, 16)] = (
                        rowbuf[r, pl.ds(cc * 16, 16)] * av)

            pltpu.sync_copy(rowbuf, out_sh.at[dstl_loc.at[cid * 44 + j]],
                            add=True)

        plsc.subcore_barrier()
        for c5 in range(5):
            rows = pl.ds(tid * 640 + c5 * 128, 128)
            pltpu.sync_copy(out_sh.at[rows], out_hbm.at[cid].at[h].at[rows])


@jax.jit
def _edge_phase(z, s, d, src2, dst2, iota_rows):
    kfn = pl.kernel(
        _edge_body,
        out_type=jax.ShapeDtypeStruct((2, 2, NPAD, 64), F32),
        mesh=_mesh,
        compiler_params=_sc_params,
        scratch_types=[
            pltpu.VMEM((NPAD,), F32),
            pltpu.VMEM((NPAD,), F32),
            pltpu.VMEM((88, 128), jnp.int32),
            pltpu.VMEM((88, 128), jnp.int32),
            pltpu.VMEM((NROWS, 128), F32),
            pltpu.VMEM((EW32,), F32),
            pltpu.VMEM((128, 64), F32),
            pltpu.VMEM((16,), F32),
            pltpu.VMEM((16, 16), F32),
            pltpu.VMEM((1, NROWS), jnp.int32),
            pltpu.VMEM_SHARED((NPAD, 64), F32),
            pltpu.VMEM_SHARED((NROWS, 128), F32),
            pltpu.VMEM_SHARED((16, 16), F32),
        ],
    )
    return kfn(z, s, d, src2, dst2, iota_rows)


def _gather_body(h_hbm, idx_hbm, g_hbm, idx_loc, rowbuf):
    cid = lax.axis_index("c")
    tid = lax.axis_index("s")
    wid = tid * 2 + cid
    pltpu.sync_copy(idx_hbm.at[wid], idx_loc)
    for c in range(5):
        pltpu.sync_copy(h_hbm.at[idx_loc.at[c]], rowbuf)
        pltpu.sync_copy(rowbuf, g_hbm.at[pl.ds(wid * 320 + c * 64, 64)])


@jax.jit
def _cluster_gather(h2, idx):
    kfn = pl.kernel(
        _gather_body,
        out_type=jax.ShapeDtypeStruct((NPAD, D), F32),
        mesh=_mesh,
        scratch_types=[
            pltpu.VMEM((5, 64), jnp.int32),
            pltpu.VMEM((64, 128), F32),
        ],
    )
    return kfn(h2, idx)


_ROWS_BLK = 256
_N_BLKS = NPAD // _ROWS_BLK


def _proj1_body(x_ref, wi_ref, bi_ref, w_ref, asrc_ref, adst_ref,
                z_ref, s_ref, d_ref):
    h = jnp.dot(x_ref[...], wi_ref[...], preferred_element_type=F32) + bi_ref[...]
    z = jnp.dot(h, w_ref[...], preferred_element_type=F32)
    z_ref[...] = z
    s_ref[...] = jnp.dot(z, asrc_ref[...], preferred_element_type=F32)
    d_ref[...] = jnp.dot(z, adst_ref[...], preferred_element_type=F32)


@jax.jit
def _proj1(x, w_in, b_in, w, a_src, a_dst):
    full = lambda i: (0, 0)
    return pl.pallas_call(
        _proj1_body,
        grid=(_N_BLKS,),
        in_specs=[
            pl.BlockSpec((_ROWS_BLK, D), lambda i: (i, 0)),
            pl.BlockSpec((D, D), full),
            pl.BlockSpec((1, D), full),
            pl.BlockSpec((D, D), full),
            pl.BlockSpec((D, 1), full),
            pl.BlockSpec((D, 1), full),
        ],
        out_specs=[
            pl.BlockSpec((_ROWS_BLK, D), lambda i: (i, 0)),
            pl.BlockSpec((_ROWS_BLK, 1), lambda i: (i, 0)),
            pl.BlockSpec((_ROWS_BLK, 1), lambda i: (i, 0)),
        ],
        out_shape=[
            jax.ShapeDtypeStruct((NPAD, D), F32),
            jax.ShapeDtypeStruct((NPAD, 1), F32),
            jax.ShapeDtypeStruct((NPAD, 1), F32),
        ],
    )(x, w_in, b_in[None, :], w, a_src[:, None], a_dst[:, None])


def _proj2_body(p_ref, b_ref, w_ref, asrc_ref, adst_ref, z_ref, s_ref, d_ref):
    h = jnp.concatenate([p_ref[0, 0] + p_ref[1, 0],
                         p_ref[0, 1] + p_ref[1, 1]], axis=-1) + b_ref[...]
    z = jnp.dot(h, w_ref[...], preferred_element_type=F32)
    z_ref[...] = z
    s_ref[...] = jnp.dot(z, asrc_ref[...], preferred_element_type=F32)
    d_ref[...] = jnp.dot(z, adst_ref[...], preferred_element_type=F32)


@jax.jit
def _proj2(p, b, w, a_src, a_dst):
    full = lambda i: (0, 0)
    return pl.pallas_call(
        _proj2_body,
        grid=(_N_BLKS,),
        in_specs=[
            pl.BlockSpec((2, 2, _ROWS_BLK, 64), lambda i: (0, 0, i, 0)),
            pl.BlockSpec((1, D), full),
            pl.BlockSpec((D, D), full),
            pl.BlockSpec((D, 1), full),
            pl.BlockSpec((D, 1), full),
        ],
        out_specs=[
            pl.BlockSpec((_ROWS_BLK, D), lambda i: (i, 0)),
            pl.BlockSpec((_ROWS_BLK, 1), lambda i: (i, 0)),
            pl.BlockSpec((_ROWS_BLK, 1), lambda i: (i, 0)),
        ],
        out_shape=[
            jax.ShapeDtypeStruct((NPAD, D), F32),
            jax.ShapeDtypeStruct((NPAD, 1), F32),
            jax.ShapeDtypeStruct((NPAD, 1), F32),
        ],
    )(p, b[None, :], w, a_src[:, None], a_dst[:, None])


def _sum2_body(p_ref, b_ref, h_ref):
    h_ref[...] = jnp.concatenate([p_ref[0, 0] + p_ref[1, 0],
                                  p_ref[0, 1] + p_ref[1, 1]],
                                 axis=-1) + b_ref[...]


@jax.jit
def _sum2(p, b):
    return pl.pallas_call(
        _sum2_body,
        grid=(_N_BLKS,),
        in_specs=[
            pl.BlockSpec((2, 2, _ROWS_BLK, 64), lambda i: (0, 0, i, 0)),
            pl.BlockSpec((1, D), lambda i: (0, 0)),
        ],
        out_specs=pl.BlockSpec((_ROWS_BLK, D), lambda i: (i, 0)),
        out_shape=jax.ShapeDtypeStruct((NPAD, D), F32),
    )(p, b[None, :])


def _emb_body(g_ref, m_ref, w_ref, b_ref, ce_ref, acc):
    k = pl.program_id(0)

    @pl.when(k == 0)
    def _():
        acc[...] = jnp.zeros_like(acc)

    acc[...] += jnp.dot(g_ref[0] * m_ref[0], w_ref[0],
                        preferred_element_type=F32)

    @pl.when(k == K - 1)
    def _():
        ce_ref[...] = acc[...] + b_ref[...]


@jax.jit
def _emb(g, m, w, b):
    return pl.pallas_call(
        _emb_body,
        grid=(K,),
        in_specs=[
            pl.BlockSpec((1, SPAD, D), lambda k: (k, 0, 0)),
            pl.BlockSpec((1, SPAD, D), lambda k: (k, 0, 0)),
            pl.BlockSpec((1, D, D), lambda k: (k, 0, 0)),
            pl.BlockSpec((1, D), lambda k: (0, 0)),
        ],
        out_specs=pl.BlockSpec((SPAD, D), lambda k: (0, 0)),
        out_shape=jax.ShapeDtypeStruct((SPAD, D), F32),
        scratch_shapes=[pltpu.VMEM((SPAD, D), F32)],
    )(g, m, w, b[None, :])


def _ln(x, s, b):
    m = x.mean(-1, keepdims=True)
    v = ((x - m) ** 2).mean(-1, keepdims=True)
    return (x - m) / jnp.sqrt(v + 1e-5) * s + b


def _tf_body(y0_ref, l1s, l1b, wq, bq, wk, bk, wv, bv, wo, bo,
             l2s, l2b, wf1, bf1, wf2, bf2, wcls, bcls, out_ref, y):
    i = pl.program_id(0)

    @pl.when(i == 0)
    def _():
        y[...] = y0_ref[...]

    hd = D // HEADS
    colmask = lax.broadcasted_iota(jnp.int32, (SPAD, SPAD), 1) < (C + 1)

    t = _ln(y[...], l1s[0], l1b[0])
    q = jnp.dot(t, wq[0], preferred_element_type=F32) + bq[0]
    kk = jnp.dot(t, wk[0], preferred_element_type=F32) + bk[0]
    v = jnp.dot(t, wv[0], preferred_element_type=F32) + bv[0]
    ohs = []
    for h in range(HEADS):
        qh = q[:, h * hd:(h + 1) * hd]
        kh = kk[:, h * hd:(h + 1) * hd]
        vh = v[:, h * hd:(h + 1) * hd]
        sc = lax.dot_general(qh, kh, (((1,), (1,)), ((), ())),
                             preferred_element_type=F32) * (1.0 / 4.0)
        sc = jnp.where(colmask, sc, -1e30)
        sc = sc - sc.max(-1, keepdims=True)
        ex = jnp.exp(sc)
        att = ex / ex.sum(-1, keepdims=True)
        ohs.append(jnp.dot(att, vh, preferred_element_type=F32))
    o = jnp.concatenate(ohs, axis=1)
    ynew = y[...] + jnp.dot(o, wo[0], preferred_element_type=F32) + bo[0]
    t2 = _ln(ynew, l2s[0], l2b[0])
    f = jnp.dot(jax.nn.gelu(jnp.dot(t2, wf1[0], preferred_element_type=F32)
                            + bf1[0]),
                wf2[0], preferred_element_type=F32) + bf2[0]
    y[...] = ynew + f

    @pl.when(i == DEPTH - 1)
    def _():
        out_ref[...] = (jnp.dot(y[0:1, :], wcls[...],
                                preferred_element_type=F32) + bcls[...])


@jax.jit
def _transformer(y0, p):
    cube = lambda i: (i, 0, 0)
    fixed = lambda i: (0, 0)
    row3 = lambda n: pl.BlockSpec((1, 1, n), cube)
    r2 = lambda a: a[:, None, :]
    return pl.pallas_call(
        _tf_body,
        grid=(DEPTH,),
        in_specs=[
            pl.BlockSpec((SPAD, D), fixed),
            row3(D), row3(D),
            pl.BlockSpec((1, D, D), cube), row3(D),
            pl.BlockSpec((1, D, D), cube), row3(D),
            pl.BlockSpec((1, D, D), cube), row3(D),
            pl.BlockSpec((1, D, D), cube), row3(D),
            row3(D), row3(D),
            pl.BlockSpec((1, D, 4 * D), cube), row3(4 * D),
            pl.BlockSpec((1, 4 * D, D), cube), row3(D),
            pl.BlockSpec((D, NUM_CLASSES), fixed),
            pl.BlockSpec((1, NUM_CLASSES), fixed),
        ],
        out_specs=pl.BlockSpec((1, NUM_CLASSES), fixed),
        out_shape=jax.ShapeDtypeStruct((1, NUM_CLASSES), F32),
        scratch_shapes=[pltpu.VMEM((SPAD, D), F32)],
    )(y0, r2(p['ln1_s']), r2(p['ln1_b']), p['Wq'], r2(p['bq']),
      p['Wk'], r2(p['bk']), p['Wv'], r2(p['bv']), p['Wo'], r2(p['bo']),
      r2(p['ln2_s']), r2(p['ln2_b']),
      p['Wf1'], r2(p['bf1']), p['Wf2'], r2(p['bf2']),
      p['W_cls'], p['b_cls'][None, :])


def kernel(x, params, edge_index, cluster_indices, cluster_mask):
    p = params
    xp = jnp.pad(x, ((0, NPAD - N), (0, 0)))
    sl = jnp.arange(N, dtype=jnp.int32)
    npad_e = EPAD - E - N
    src = jnp.concatenate([edge_index[0].astype(jnp.int32), sl,
                           jnp.full((npad_e,), DUMMY, jnp.int32)])
    dst = jnp.concatenate([edge_index[1].astype(jnp.int32), sl,
                           jnp.full((npad_e,), DUMMY, jnp.int32)])
    src2 = src.reshape(32, 44, 128)
    dst2 = dst.reshape(32, 44, 128)
    iota_rows = jnp.arange(NROWS, dtype=jnp.int32).reshape(1, NROWS)

    halves = lambda z: jnp.stack([z[:, :64], z[:, 64:]])
    z1, s1, d1 = _proj1(xp, p['W_in'], p['b_in'], p['W1'],
                        p['a_src1'], p['a_dst1'])
    pp1 = _edge_phase(halves(z1), s1.reshape(-1), d1.reshape(-1),
                      src2, dst2, iota_rows)
    z2, s2, d2 = _proj2(pp1, p['b1'], p['W2'], p['a_src2'], p['a_dst2'])
    pp2 = _edge_phase(halves(z2), s2.reshape(-1), d2.reshape(-1),
                      src2, dst2, iota_rows)
    h2 = _sum2(pp2, p['b2'])

    cit = jnp.pad(cluster_indices.T.astype(jnp.int32), ((0, 0), (0, 7)))
    idx = cit.reshape(32, 5, 64)
    mt = jnp.pad(cluster_mask.T, ((0, 0), (0, 7)))
    mbc = jnp.broadcast_to(mt[:, :, None], (K, SPAD, D))

    g = _cluster_gather(h2, idx)
    ce = _emb(g.reshape(K, SPAD, D), mbc, p['W_emb'].reshape(K, D, D),
              p['b_emb'])
    y0 = jnp.concatenate([p['class_token'][None, :], ce[:C],
                          jnp.zeros((SPAD - C - 1, D), F32)], axis=0)
    return _transformer(y0, p)


# speedup vs baseline: 7.5294x; 1.0162x over previous
import dataclasses
import functools

import jax
import jax.numpy as jnp
from jax import lax
from jax.experimental import pallas as pl
from jax.experimental.pallas import tpu as pltpu
from jax.experimental.pallas import tpu_sc as plsc

N = 10000
E = 160000
D = 128
K = 32
C = 313
HEADS = 8
DEPTH = 8
NUM_CLASSES = 10

NPAD = 10240
EPAD = 180224
EW32 = EPAD // 32
EW16 = EPAD // 16
NROWS = NPAD // 128
DUMMY = N + 64
SPAD = 320
F32 = jnp.float32

_mesh = plsc.VectorSubcoreMesh(core_axis_name="c", subcore_axis_name="s")

_sc_params = pltpu.CompilerParams(use_tc_tiling_on_sc=False)
if "needs_layout_passes" in pltpu.CompilerParams.__dataclass_fields__:
    _sc_params = dataclasses.replace(_sc_params, needs_layout_passes=False)


def _edge_body(z_hbm, s_hbm, d_hbm, src2_hbm, dst2_hbm,
               iota_hbm, out_hbm,
               s_loc, d_loc, srcl_loc, dstl_loc, ebuf,
               den_loc, alpha_loc, gbuf0, gbuf1, sbuf0, sbuf1,
               mx_loc, mred_loc, iota_loc,
               out_sh, den_sh, mx_sh,
               gsem0, gsem1, ssem0, ssem1):
    cid = lax.axis_index("c")
    tid = lax.axis_index("s")

    pltpu.sync_copy(s_hbm, s_loc)
    pltpu.sync_copy(d_hbm, d_loc)
    pltpu.sync_copy(src2_hbm.at[2 * tid], srcl_loc.at[pl.ds(0, 44)])
    pltpu.sync_copy(src2_hbm.at[2 * tid + 1], srcl_loc.at[pl.ds(44, 44)])
    pltpu.sync_copy(dst2_hbm.at[2 * tid], dstl_loc.at[pl.ds(0, 44)])
    pltpu.sync_copy(dst2_hbm.at[2 * tid + 1], dstl_loc.at[pl.ds(44, 44)])
    pltpu.sync_copy(iota_hbm, iota_loc)

    zero16 = jnp.zeros((16,), F32)

    @pl.loop(0, NROWS)
    def _(r):
        for cc in range(8):
            den_loc[r, pl.ds(cc * 16, 16)] = zero16

    @pl.when(tid == 0)
    def _():
        pltpu.sync_copy(den_loc, den_sh)

    mx_loc[...] = jnp.full((16,), -3e38, F32)

    @pl.loop(0, 88)
    def _(r):
        for cc in range(8):
            si = srcl_loc[r, pl.ds(cc * 16, 16)]
            di = dstl_loc[r, pl.ds(cc * 16, 16)]
            ev = plsc.load_gather(s_loc, [si]) + plsc.load_gather(d_loc, [di])
            ev = jnp.where(ev > 0, ev, 0.2 * ev)
            ebuf[r, pl.ds(cc * 16, 16)] = ev
            mx_loc[...] = jnp.maximum(mx_loc[...], ev)

    pltpu.sync_copy(mx_loc, mx_sh.at[tid])
    plsc.subcore_barrier()
    pltpu.sync_copy(mx_sh, mred_loc)
    mv = mred_loc[0, :]
    for t in range(1, 16):
        mv = jnp.maximum(mv, mred_loc[t, :])
    mscal = jnp.max(mv)
    mvec = jnp.full((16,), mscal, F32)

    @pl.loop(0, 88)
    def _(r):
        for cc in range(8):
            di = dstl_loc[r, pl.ds(cc * 16, 16)]
            ex = jnp.exp(ebuf[r, pl.ds(cc * 16, 16)] - mvec)
            plsc.addupdate_scatter(den_loc, [di >> 7, di & 127], ex)

    pltpu.sync_copy(den_loc, den_sh.at[iota_loc.at[0]], add=True)
    plsc.subcore_barrier()
    pltpu.sync_copy(den_sh, den_loc)

    @pl.loop(0, 44)
    def _(r):
        for cc in range(8):
            di = dstl_loc[cid * 44 + r, pl.ds(cc * 16, 16)]
            ex = jnp.exp(ebuf[cid * 44 + r, pl.ds(cc * 16, 16)] - mvec)
            dn = plsc.load_gather(den_loc, [di >> 7, di & 127])
            alpha_loc[pl.ds(r * 128 + cc * 16, 16)] = ex / (dn + 1e-16)

    gbufs = (gbuf0, gbuf1)
    sbufs = (sbuf0, sbuf1)
    gsems = (gsem0, gsem1)
    ssems = (ssem0, ssem1)
    for h in range(4):
        @pl.loop(0, 128)
        def _(r):
            for cc in range(2):
                gbuf0[r, pl.ds(cc * 16, 16)] = zero16
        for c5 in range(5):
            pltpu.sync_copy(gbuf0, out_sh.at[pl.ds(tid * 640 + c5 * 128, 128)])
        plsc.subcore_barrier()

        for b in range(2):
            pltpu.async_copy(z_hbm.at[h].at[srcl_loc.at[cid * 44 + b]],
                             gbufs[b], gsems[b])

        @pl.loop(0, 44, step=2)
        def _(jj):
            for b in range(2):
                j = jj + b
                pltpu.make_async_copy(
                    z_hbm.at[h].at[srcl_loc.at[cid * 44 + j]],
                    gbufs[b], gsems[b]).wait()

                @pl.when(jj > 0)
                def _():
                    pltpu.make_async_copy(
                        sbufs[b], out_sh.at[dstl_loc.at[cid * 44 + j]],
                        ssems[b]).wait()

                @pl.loop(0, 128)
                def _(r):
                    av = plsc.load_gather(
                        alpha_loc, [jnp.full((16,), j * 128 + r, jnp.int32)])
                    for cc in range(2):
                        sbufs[b][r, pl.ds(cc * 16, 16)] = (
                            gbufs[b][r, pl.ds(cc * 16, 16)] * av)

                pltpu.async_copy(sbufs[b],
                                 out_sh.at[dstl_loc.at[cid * 44 + j]],
                                 ssems[b], add=True)

                @pl.when(j + 2 < 44)
                def _():
                    pltpu.async_copy(
                        z_hbm.at[h].at[srcl_loc.at[cid * 44 + j + 2]],
                        gbufs[b], gsems[b])

        for b in range(2):
            pltpu.make_async_copy(sbufs[b],
                                  out_sh.at[dstl_loc.at[cid * 44]],
                                  ssems[b]).wait()

        plsc.subcore_barrier()
        for c5 in range(5):
            rows = pl.ds(tid * 640 + c5 * 128, 128)
            pltpu.sync_copy(out_sh.at[rows], out_hbm.at[cid].at[h].at[rows])


@jax.jit
def _edge_phase(z, s, d, src2, dst2, iota_rows):
    kfn = pl.kernel(
        _edge_body,
        out_type=jax.ShapeDtypeStruct((2, 4, NPAD, 32), F32),
        mesh=_mesh,
        compiler_params=_sc_params,
        scratch_types=[
            pltpu.VMEM((NPAD,), F32),
            pltpu.VMEM((NPAD,), F32),
            pltpu.VMEM((88, 128), jnp.int32),
            pltpu.VMEM((88, 128), jnp.int32),
            pltpu.VMEM((88, 128), F32),
            pltpu.VMEM((NROWS, 128), F32),
            pltpu.VMEM((EW32,), F32),
            pltpu.VMEM((128, 32), F32),
            pltpu.VMEM((128, 32), F32),
            pltpu.VMEM((128, 32), F32),
            pltpu.VMEM((128, 32), F32),
            pltpu.VMEM((16,), F32),
            pltpu.VMEM((16, 16), F32),
            pltpu.VMEM((1, NROWS), jnp.int32),
            pltpu.VMEM_SHARED((NPAD, 32), F32),
            pltpu.VMEM_SHARED((NROWS, 128), F32),
            pltpu.VMEM_SHARED((16, 16), F32),
            pltpu.SemaphoreType.DMA,
            pltpu.SemaphoreType.DMA,
            pltpu.SemaphoreType.DMA,
            pltpu.SemaphoreType.DMA,
        ],
    )
    return kfn(z, s, d, src2, dst2, iota_rows)


def _gather_body(h_hbm, idx_hbm, g_hbm, idx_loc, rowbuf):
    cid = lax.axis_index("c")
    tid = lax.axis_index("s")
    wid = tid * 2 + cid
    pltpu.sync_copy(idx_hbm.at[wid], idx_loc)
    for c in range(5):
        pltpu.sync_copy(h_hbm.at[idx_loc.at[c]], rowbuf)
        pltpu.sync_copy(rowbuf, g_hbm.at[pl.ds(wid * 320 + c * 64, 64)])


@jax.jit
def _cluster_gather(h2, idx):
    kfn = pl.kernel(
        _gather_body,
        out_type=jax.ShapeDtypeStruct((NPAD, D), F32),
        mesh=_mesh,
        scratch_types=[
            pltpu.VMEM((5, 64), jnp.int32),
            pltpu.VMEM((64, 128), F32),
        ],
    )
    return kfn(h2, idx)


_ROWS_BLK = 256
_N_BLKS = NPAD // _ROWS_BLK


def _proj1_body(x_ref, wi_ref, bi_ref, w_ref, asrc_ref, adst_ref,
                z_ref, s_ref, d_ref):
    h = jnp.dot(x_ref[...], wi_ref[...], preferred_element_type=F32) + bi_ref[...]
    z = jnp.dot(h, w_ref[...], preferred_element_type=F32)
    z_ref[...] = z
    s_ref[...] = jnp.dot(z, asrc_ref[...], preferred_element_type=F32)
    d_ref[...] = jnp.dot(z, adst_ref[...], preferred_element_type=F32)


@jax.jit
def _proj1(x, w_in, b_in, w, a_src, a_dst):
    full = lambda i: (0, 0)
    return pl.pallas_call(
        _proj1_body,
        grid=(_N_BLKS,),
        in_specs=[
            pl.BlockSpec((_ROWS_BLK, D), lambda i: (i, 0)),
            pl.BlockSpec((D, D), full),
            pl.BlockSpec((1, D), full),
            pl.BlockSpec((D, D), full),
            pl.BlockSpec((D, 1), full),
            pl.BlockSpec((D, 1), full),
        ],
        out_specs=[
            pl.BlockSpec((_ROWS_BLK, D), lambda i: (i, 0)),
            pl.BlockSpec((_ROWS_BLK, 1), lambda i: (i, 0)),
            pl.BlockSpec((_ROWS_BLK, 1), lambda i: (i, 0)),
        ],
        out_shape=[
            jax.ShapeDtypeStruct((NPAD, D), F32),
            jax.ShapeDtypeStruct((NPAD, 1), F32),
            jax.ShapeDtypeStruct((NPAD, 1), F32),
        ],
    )(x, w_in, b_in[None, :], w, a_src[:, None], a_dst[:, None])


def _proj2_body(p_ref, b_ref, w_ref, asrc_ref, adst_ref, z_ref, s_ref, d_ref):
    h = jnp.concatenate([p_ref[0, q] + p_ref[1, q] for q in range(4)],
                        axis=-1) + b_ref[...]
    z = jnp.dot(h, w_ref[...], preferred_element_type=F32)
    z_ref[...] = z
    s_ref[...] = jnp.dot(z, asrc_ref[...], preferred_element_type=F32)
    d_ref[...] = jnp.dot(z, adst_ref[...], preferred_element_type=F32)


@jax.jit
def _proj2(p, b, w, a_src, a_dst):
    full = lambda i: (0, 0)
    return pl.pallas_call(
        _proj2_body,
        grid=(_N_BLKS,),
        in_specs=[
            pl.BlockSpec((2, 4, _ROWS_BLK, 32), lambda i: (0, 0, i, 0)),
            pl.BlockSpec((1, D), full),
            pl.BlockSpec((D, D), full),
            pl.BlockSpec((D, 1), full),
            pl.BlockSpec((D, 1), full),
        ],
        out_specs=[
            pl.BlockSpec((_ROWS_BLK, D), lambda i: (i, 0)),
            pl.BlockSpec((_ROWS_BLK, 1), lambda i: (i, 0)),
            pl.BlockSpec((_ROWS_BLK, 1), lambda i: (i, 0)),
        ],
        out_shape=[
            jax.ShapeDtypeStruct((NPAD, D), F32),
            jax.ShapeDtypeStruct((NPAD, 1), F32),
            jax.ShapeDtypeStruct((NPAD, 1), F32),
        ],
    )(p, b[None, :], w, a_src[:, None], a_dst[:, None])


def _sum2_body(p_ref, b_ref, h_ref):
    h_ref[...] = jnp.concatenate([p_ref[0, q] + p_ref[1, q]
                                  for q in range(4)], axis=-1) + b_ref[...]


@jax.jit
def _sum2(p, b):
    return pl.pallas_call(
        _sum2_body,
        grid=(_N_BLKS,),
        in_specs=[
            pl.BlockSpec((2, 4, _ROWS_BLK, 32), lambda i: (0, 0, i, 0)),
            pl.BlockSpec((1, D), lambda i: (0, 0)),
        ],
        out_specs=pl.BlockSpec((_ROWS_BLK, D), lambda i: (i, 0)),
        out_shape=jax.ShapeDtypeStruct((NPAD, D), F32),
    )(p, b[None, :])


def _emb_body(g_ref, m_ref, w_ref, b_ref, ce_ref, acc):
    k = pl.program_id(0)

    @pl.when(k == 0)
    def _():
        acc[...] = jnp.zeros_like(acc)

    acc[...] += jnp.dot(g_ref[0] * m_ref[0], w_ref[0],
                        preferred_element_type=F32)

    @pl.when(k == K - 1)
    def _():
        ce_ref[...] = acc[...] + b_ref[...]


@jax.jit
def _emb(g, m, w, b):
    return pl.pallas_call(
        _emb_body,
        grid=(K,),
        in_specs=[
            pl.BlockSpec((1, SPAD, D), lambda k: (k, 0, 0)),
            pl.BlockSpec((1, SPAD, D), lambda k: (k, 0, 0)),
            pl.BlockSpec((1, D, D), lambda k: (k, 0, 0)),
            pl.BlockSpec((1, D), lambda k: (0, 0)),
        ],
        out_specs=pl.BlockSpec((SPAD, D), lambda k: (0, 0)),
        out_shape=jax.ShapeDtypeStruct((SPAD, D), F32),
        scratch_shapes=[pltpu.VMEM((SPAD, D), F32)],
    )(g, m, w, b[None, :])


def _ln(x, s, b):
    m = x.mean(-1, keepdims=True)
    v = ((x - m) ** 2).mean(-1, keepdims=True)
    return (x - m) / jnp.sqrt(v + 1e-5) * s + b


def _tf_body(y0_ref, l1s, l1b, wq, bq, wk, bk, wv, bv, wo, bo,
             l2s, l2b, wf1, bf1, wf2, bf2, wcls, bcls, out_ref, y):
    i = pl.program_id(0)

    @pl.when(i == 0)
    def _():
        y[...] = y0_ref[...]

    hd = D // HEADS
    colmask = lax.broadcasted_iota(jnp.int32, (SPAD, SPAD), 1) < (C + 1)

    t = _ln(y[...], l1s[0], l1b[0])
    q = jnp.dot(t, wq[0], preferred_element_type=F32) + bq[0]
    kk = jnp.dot(t, wk[0], preferred_element_type=F32) + bk[0]
    v = jnp.dot(t, wv[0], preferred_element_type=F32) + bv[0]
    ohs = []
    for h in range(HEADS):
        qh = q[:, h * hd:(h + 1) * hd]
        kh = kk[:, h * hd:(h + 1) * hd]
        vh = v[:, h * hd:(h + 1) * hd]
        sc = lax.dot_general(qh, kh, (((1,), (1,)), ((), ())),
                             preferred_element_type=F32) * (1.0 / 4.0)
        sc = jnp.where(colmask, sc, -1e30)
        sc = sc - sc.max(-1, keepdims=True)
        ex = jnp.exp(sc)
        att = ex / ex.sum(-1, keepdims=True)
        ohs.append(jnp.dot(att, vh, preferred_element_type=F32))
    o = jnp.concatenate(ohs, axis=1)
    ynew = y[...] + jnp.dot(o, wo[0], preferred_element_type=F32) + bo[0]
    t2 = _ln(ynew, l2s[0], l2b[0])
    f = jnp.dot(jax.nn.gelu(jnp.dot(t2, wf1[0], preferred_element_type=F32)
                            + bf1[0]),
                wf2[0], preferred_element_type=F32) + bf2[0]
    y[...] = ynew + f

    @pl.when(i == DEPTH - 1)
    def _():
        out_ref[...] = (jnp.dot(y[0:1, :], wcls[...],
                                preferred_element_type=F32) + bcls[...])


@jax.jit
def _transformer(y0, p):
    cube = lambda i: (i, 0, 0)
    fixed = lambda i: (0, 0)
    row3 = lambda n: pl.BlockSpec((1, 1, n), cube)
    r2 = lambda a: a[:, None, :]
    return pl.pallas_call(
        _tf_body,
        grid=(DEPTH,),
        in_specs=[
            pl.BlockSpec((SPAD, D), fixed),
            row3(D), row3(D),
            pl.BlockSpec((1, D, D), cube), row3(D),
            pl.BlockSpec((1, D, D), cube), row3(D),
            pl.BlockSpec((1, D, D), cube), row3(D),
            pl.BlockSpec((1, D, D), cube), row3(D),
            row3(D), row3(D),
            pl.BlockSpec((1, D, 4 * D), cube), row3(4 * D),
            pl.BlockSpec((1, 4 * D, D), cube), row3(D),
            pl.BlockSpec((D, NUM_CLASSES), fixed),
            pl.BlockSpec((1, NUM_CLASSES), fixed),
        ],
        out_specs=pl.BlockSpec((1, NUM_CLASSES), fixed),
        out_shape=jax.ShapeDtypeStruct((1, NUM_CLASSES), F32),
        scratch_shapes=[pltpu.VMEM((SPAD, D), F32)],
    )(y0, r2(p['ln1_s']), r2(p['ln1_b']), p['Wq'], r2(p['bq']),
      p['Wk'], r2(p['bk']), p['Wv'], r2(p['bv']), p['Wo'], r2(p['bo']),
      r2(p['ln2_s']), r2(p['ln2_b']),
      p['Wf1'], r2(p['bf1']), p['Wf2'], r2(p['bf2']),
      p['W_cls'], p['b_cls'][None, :])


def kernel(x, params, edge_index, cluster_indices, cluster_mask):
    p = params
    xp = jnp.pad(x, ((0, NPAD - N), (0, 0)))
    sl = jnp.arange(N, dtype=jnp.int32)
    npad_e = EPAD - E - N
    src = jnp.concatenate([edge_index[0].astype(jnp.int32), sl,
                           jnp.full((npad_e,), DUMMY, jnp.int32)])
    dst = jnp.concatenate([edge_index[1].astype(jnp.int32), sl,
                           jnp.full((npad_e,), DUMMY, jnp.int32)])
    src2 = src.reshape(32, 44, 128)
    dst2 = dst.reshape(32, 44, 128)
    iota_rows = jnp.arange(NROWS, dtype=jnp.int32).reshape(1, NROWS)

    halves = lambda z: jnp.stack([z[:, 32 * q:32 * q + 32] for q in range(4)])
    z1, s1, d1 = _proj1(xp, p['W_in'], p['b_in'], p['W1'],
                        p['a_src1'], p['a_dst1'])
    pp1 = _edge_phase(halves(z1), s1.reshape(-1), d1.reshape(-1),
                      src2, dst2, iota_rows)
    z2, s2, d2 = _proj2(pp1, p['b1'], p['W2'], p['a_src2'], p['a_dst2'])
    pp2 = _edge_phase(halves(z2), s2.reshape(-1), d2.reshape(-1),
                      src2, dst2, iota_rows)
    h2 = _sum2(pp2, p['b2'])

    cit = jnp.pad(cluster_indices.T.astype(jnp.int32), ((0, 0), (0, 7)))
    idx = cit.reshape(32, 5, 64)
    mt = jnp.pad(cluster_mask.T, ((0, 0), (0, 7)))
    mbc = jnp.broadcast_to(mt[:, :, None], (K, SPAD, D))

    g = _cluster_gather(h2, idx)
    ce = _emb(g.reshape(K, SPAD, D), mbc, p['W_emb'].reshape(K, D, D),
              p['b_emb'])
    y0 = jnp.concatenate([p['class_token'][None, :], ce[:C],
                          jnp.zeros((SPAD - C - 1, D), F32)], axis=0)
    return _transformer(y0, p)


# speedup vs baseline: 7.6674x; 1.0183x over previous
import dataclasses
import functools

import jax
import jax.numpy as jnp
from jax import lax
from jax.experimental import pallas as pl
from jax.experimental.pallas import tpu as pltpu
from jax.experimental.pallas import tpu_sc as plsc

N = 10000
E = 160000
D = 128
K = 32
C = 313
HEADS = 8
DEPTH = 8
NUM_CLASSES = 10

NPAD = 10240
EPAD = 180224
EW32 = EPAD // 32
EW16 = EPAD // 16
NROWS = NPAD // 128
DUMMY = N + 64
SPAD = 320
F32 = jnp.float32

_mesh = plsc.VectorSubcoreMesh(core_axis_name="c", subcore_axis_name="s")

_sc_params = pltpu.CompilerParams(use_tc_tiling_on_sc=False)
if "needs_layout_passes" in pltpu.CompilerParams.__dataclass_fields__:
    _sc_params = dataclasses.replace(_sc_params, needs_layout_passes=False)


def _edge_body(z_hbm, s_hbm, d_hbm, src2_hbm, dst2_hbm,
               iota_hbm, out_hbm,
               s_loc, d_loc, srcl_loc, dstl_loc, ebuf,
               den_loc, alpha_loc, gbuf0, gbuf1, sbuf0, sbuf1,
               mx_loc, mred_loc, iota_loc,
               out_sh, den_sh, mx_sh,
               gsem0, gsem1, ssem0, ssem1):
    cid = lax.axis_index("c")
    tid = lax.axis_index("s")

    pltpu.sync_copy(s_hbm, s_loc)
    pltpu.sync_copy(d_hbm, d_loc)
    pltpu.sync_copy(src2_hbm.at[2 * tid], srcl_loc.at[pl.ds(0, 44)])
    pltpu.sync_copy(src2_hbm.at[2 * tid + 1], srcl_loc.at[pl.ds(44, 44)])
    pltpu.sync_copy(dst2_hbm.at[2 * tid], dstl_loc.at[pl.ds(0, 44)])
    pltpu.sync_copy(dst2_hbm.at[2 * tid + 1], dstl_loc.at[pl.ds(44, 44)])
    pltpu.sync_copy(iota_hbm, iota_loc)

    zero16 = jnp.zeros((16,), F32)

    @pl.loop(0, NROWS)
    def _(r):
        for cc in range(8):
            den_loc[r, pl.ds(cc * 16, 16)] = zero16

    @pl.when(tid == 0)
    def _():
        pltpu.sync_copy(den_loc, den_sh)

    mx_loc[...] = jnp.full((16,), -3e38, F32)

    @pl.loop(0, 88)
    def _(r):
        for cc in range(8):
            si = srcl_loc[r, pl.ds(cc * 16, 16)]
            di = dstl_loc[r, pl.ds(cc * 16, 16)]
            ev = plsc.load_gather(s_loc, [si]) + plsc.load_gather(d_loc, [di])
            ev = jnp.where(ev > 0, ev, 0.2 * ev)
            ebuf[r, pl.ds(cc * 16, 16)] = ev
            mx_loc[...] = jnp.maximum(mx_loc[...], ev)

    pltpu.sync_copy(mx_loc, mx_sh.at[tid])
    plsc.subcore_barrier()
    pltpu.sync_copy(mx_sh, mred_loc)
    mv = mred_loc[0, :]
    for t in range(1, 16):
        mv = jnp.maximum(mv, mred_loc[t, :])
    mscal = jnp.max(mv)
    mvec = jnp.full((16,), mscal, F32)

    @pl.loop(0, 88)
    def _(r):
        for cc in range(8):
            di = dstl_loc[r, pl.ds(cc * 16, 16)]
            ex = jnp.exp(ebuf[r, pl.ds(cc * 16, 16)] - mvec)
            plsc.addupdate_scatter(den_loc, [di >> 7, di & 127], ex)

    pltpu.sync_copy(den_loc, den_sh.at[iota_loc.at[0]], add=True)
    plsc.subcore_barrier()
    pltpu.sync_copy(den_sh, den_loc)

    @pl.loop(0, 44)
    def _(r):
        for cc in range(8):
            di = dstl_loc[cid * 44 + r, pl.ds(cc * 16, 16)]
            ex = jnp.exp(ebuf[cid * 44 + r, pl.ds(cc * 16, 16)] - mvec)
            dn = plsc.load_gather(den_loc, [di >> 7, di & 127])
            alpha_loc[pl.ds(r * 128 + cc * 16, 16)] = ex / (dn + 1e-16)

    gbufs = (gbuf0, gbuf1)
    sbufs = (sbuf0, sbuf1)
    gsems = (gsem0, gsem1)
    ssems = (ssem0, ssem1)
    for h in range(4):
        @pl.loop(0, 128)
        def _(r):
            for cc in range(2):
                gbuf0[r, pl.ds(cc * 16, 16)] = zero16
        for c5 in range(5):
            pltpu.sync_copy(gbuf0, out_sh.at[pl.ds(tid * 640 + c5 * 128, 128)])
        plsc.subcore_barrier()

        for b in range(2):
            pltpu.async_copy(z_hbm.at[h].at[srcl_loc.at[cid * 44 + b]],
                             gbufs[b], gsems[b])

        @pl.loop(0, 44, step=2)
        def _(jj):
            for b in range(2):
                j = jj + b
                pltpu.make_async_copy(
                    z_hbm.at[h].at[srcl_loc.at[cid * 44 + j]],
                    gbufs[b], gsems[b]).wait()

                @pl.when(jj > 0)
                def _():
                    pltpu.make_async_copy(
                        sbufs[b], out_sh.at[dstl_loc.at[cid * 44 + j]],
                        ssems[b]).wait()

                @pl.loop(0, 128, step=16)
                def _(r):
                    av16 = alpha_loc[pl.ds(j * 128 + r, 16)]
                    for k in range(16):
                        avk = lax.gather(
                            av16, jnp.full((16, 1), k, jnp.int32),
                            lax.GatherDimensionNumbers(
                                offset_dims=(), collapsed_slice_dims=(0,),
                                start_index_map=(0,)),
                            (1,), indices_are_sorted=True, unique_indices=False,
                            mode=lax.GatherScatterMode.PROMISE_IN_BOUNDS)
                        for cc in range(2):
                            sbufs[b][r + k, pl.ds(cc * 16, 16)] = (
                                gbufs[b][r + k, pl.ds(cc * 16, 16)] * avk)

                pltpu.async_copy(sbufs[b],
                                 out_sh.at[dstl_loc.at[cid * 44 + j]],
                                 ssems[b], add=True)

                @pl.when(j + 2 < 44)
                def _():
                    pltpu.async_copy(
                        z_hbm.at[h].at[srcl_loc.at[cid * 44 + j + 2]],
                        gbufs[b], gsems[b])

        for b in range(2):
            pltpu.make_async_copy(sbufs[b],
                                  out_sh.at[dstl_loc.at[cid * 44]],
                                  ssems[b]).wait()

        plsc.subcore_barrier()
        for c5 in range(5):
            rows = pl.ds(tid * 640 + c5 * 128, 128)
            pltpu.sync_copy(out_sh.at[rows], out_hbm.at[cid].at[h].at[rows])


@jax.jit
def _edge_phase(z, s, d, src2, dst2, iota_rows):
    kfn = pl.kernel(
        _edge_body,
        out_type=jax.ShapeDtypeStruct((2, 4, NPAD, 32), F32),
        mesh=_mesh,
        compiler_params=_sc_params,
        scratch_types=[
            pltpu.VMEM((NPAD,), F32),
            pltpu.VMEM((NPAD,), F32),
            pltpu.VMEM((88, 128), jnp.int32),
            pltpu.VMEM((88, 128), jnp.int32),
            pltpu.VMEM((88, 128), F32),
            pltpu.VMEM((NROWS, 128), F32),
            pltpu.VMEM((EW32,), F32),
            pltpu.VMEM((128, 32), F32),
            pltpu.VMEM((128, 32), F32),
            pltpu.VMEM((128, 32), F32),
            pltpu.VMEM((128, 32), F32),
            pltpu.VMEM((16,), F32),
            pltpu.VMEM((16, 16), F32),
            pltpu.VMEM((1, NROWS), jnp.int32),
            pltpu.VMEM_SHARED((NPAD, 32), F32),
            pltpu.VMEM_SHARED((NROWS, 128), F32),
            pltpu.VMEM_SHARED((16, 16), F32),
            pltpu.SemaphoreType.DMA,
            pltpu.SemaphoreType.DMA,
            pltpu.SemaphoreType.DMA,
            pltpu.SemaphoreType.DMA,
        ],
    )
    return kfn(z, s, d, src2, dst2, iota_rows)


def _gather_body(h_hbm, idx_hbm, g_hbm, idx_loc, rowbuf):
    cid = lax.axis_index("c")
    tid = lax.axis_index("s")
    wid = tid * 2 + cid
    pltpu.sync_copy(idx_hbm.at[wid], idx_loc)
    for c in range(5):
        pltpu.sync_copy(h_hbm.at[idx_loc.at[c]], rowbuf)
        pltpu.sync_copy(rowbuf, g_hbm.at[pl.ds(wid * 320 + c * 64, 64)])


@jax.jit
def _cluster_gather(h2, idx):
    kfn = pl.kernel(
        _gather_body,
        out_type=jax.ShapeDtypeStruct((NPAD, D), F32),
        mesh=_mesh,
        scratch_types=[
            pltpu.VMEM((5, 64), jnp.int32),
            pltpu.VMEM((64, 128), F32),
        ],
    )
    return kfn(h2, idx)


_ROWS_BLK = 256
_N_BLKS = NPAD // _ROWS_BLK


def _proj1_body(x_ref, wi_ref, bi_ref, w_ref, asrc_ref, adst_ref,
                z_ref, s_ref, d_ref):
    h = jnp.dot(x_ref[...], wi_ref[...], preferred_element_type=F32) + bi_ref[...]
    z = jnp.dot(h, w_ref[...], preferred_element_type=F32)
    z_ref[...] = z
    s_ref[...] = jnp.dot(z, asrc_ref[...], preferred_element_type=F32)
    d_ref[...] = jnp.dot(z, adst_ref[...], preferred_element_type=F32)


@jax.jit
def _proj1(x, w_in, b_in, w, a_src, a_dst):
    full = lambda i: (0, 0)
    return pl.pallas_call(
        _proj1_body,
        grid=(_N_BLKS,),
        in_specs=[
            pl.BlockSpec((_ROWS_BLK, D), lambda i: (i, 0)),
            pl.BlockSpec((D, D), full),
            pl.BlockSpec((1, D), full),
            pl.BlockSpec((D, D), full),
            pl.BlockSpec((D, 1), full),
            pl.BlockSpec((D, 1), full),
        ],
        out_specs=[
            pl.BlockSpec((_ROWS_BLK, D), lambda i: (i, 0)),
            pl.BlockSpec((_ROWS_BLK, 1), lambda i: (i, 0)),
            pl.BlockSpec((_ROWS_BLK, 1), lambda i: (i, 0)),
        ],
        out_shape=[
            jax.ShapeDtypeStruct((NPAD, D), F32),
            jax.ShapeDtypeStruct((NPAD, 1), F32),
            jax.ShapeDtypeStruct((NPAD, 1), F32),
        ],
    )(x, w_in, b_in[None, :], w, a_src[:, None], a_dst[:, None])


def _proj2_body(p_ref, b_ref, w_ref, asrc_ref, adst_ref, z_ref, s_ref, d_ref):
    h = jnp.concatenate([p_ref[0, q] + p_ref[1, q] for q in range(4)],
                        axis=-1) + b_ref[...]
    z = jnp.dot(h, w_ref[...], preferred_element_type=F32)
    z_ref[...] = z
    s_ref[...] = jnp.dot(z, asrc_ref[...], preferred_element_type=F32)
    d_ref[...] = jnp.dot(z, adst_ref[...], preferred_element_type=F32)


@jax.jit
def _proj2(p, b, w, a_src, a_dst):
    full = lambda i: (0, 0)
    return pl.pallas_call(
        _proj2_body,
        grid=(_N_BLKS,),
        in_specs=[
            pl.BlockSpec((2, 4, _ROWS_BLK, 32), lambda i: (0, 0, i, 0)),
            pl.BlockSpec((1, D), full),
            pl.BlockSpec((D, D), full),
            pl.BlockSpec((D, 1), full),
            pl.BlockSpec((D, 1), full),
        ],
        out_specs=[
            pl.BlockSpec((_ROWS_BLK, D), lambda i: (i, 0)),
            pl.BlockSpec((_ROWS_BLK, 1), lambda i: (i, 0)),
            pl.BlockSpec((_ROWS_BLK, 1), lambda i: (i, 0)),
        ],
        out_shape=[
            jax.ShapeDtypeStruct((NPAD, D), F32),
            jax.ShapeDtypeStruct((NPAD, 1), F32),
            jax.ShapeDtypeStruct((NPAD, 1), F32),
        ],
    )(p, b[None, :], w, a_src[:, None], a_dst[:, None])


def _sum2_body(p_ref, b_ref, h_ref):
    h_ref[...] = jnp.concatenate([p_ref[0, q] + p_ref[1, q]
                                  for q in range(4)], axis=-1) + b_ref[...]


@jax.jit
def _sum2(p, b):
    return pl.pallas_call(
        _sum2_body,
        grid=(_N_BLKS,),
        in_specs=[
            pl.BlockSpec((2, 4, _ROWS_BLK, 32), lambda i: (0, 0, i, 0)),
            pl.BlockSpec((1, D), lambda i: (0, 0)),
        ],
        out_specs=pl.BlockSpec((_ROWS_BLK, D), lambda i: (i, 0)),
        out_shape=jax.ShapeDtypeStruct((NPAD, D), F32),
    )(p, b[None, :])


def _emb_body(g_ref, m_ref, w_ref, b_ref, ce_ref, acc):
    k = pl.program_id(0)

    @pl.when(k == 0)
    def _():
        acc[...] = jnp.zeros_like(acc)

    acc[...] += jnp.dot(g_ref[0] * m_ref[0], w_ref[0],
                        preferred_element_type=F32)

    @pl.when(k == K - 1)
    def _():
        ce_ref[...] = acc[...] + b_ref[...]


@jax.jit
def _emb(g, m, w, b):
    return pl.pallas_call(
        _emb_body,
        grid=(K,),
        in_specs=[
            pl.BlockSpec((1, SPAD, D), lambda k: (k, 0, 0)),
            pl.BlockSpec((1, SPAD, D), lambda k: (k, 0, 0)),
            pl.BlockSpec((1, D, D), lambda k: (k, 0, 0)),
            pl.BlockSpec((1, D), lambda k: (0, 0)),
        ],
        out_specs=pl.BlockSpec((SPAD, D), lambda k: (0, 0)),
        out_shape=jax.ShapeDtypeStruct((SPAD, D), F32),
        scratch_shapes=[pltpu.VMEM((SPAD, D), F32)],
    )(g, m, w, b[None, :])


def _ln(x, s, b):
    m = x.mean(-1, keepdims=True)
    v = ((x - m) ** 2).mean(-1, keepdims=True)
    return (x - m) / jnp.sqrt(v + 1e-5) * s + b


def _tf_body(y0_ref, l1s, l1b, wq, bq, wk, bk, wv, bv, wo, bo,
             l2s, l2b, wf1, bf1, wf2, bf2, wcls, bcls, out_ref, y):
    i = pl.program_id(0)

    @pl.when(i == 0)
    def _():
        y[...] = y0_ref[...]

    hd = D // HEADS
    colmask = lax.broadcasted_iota(jnp.int32, (SPAD, SPAD), 1) < (C + 1)

    t = _ln(y[...], l1s[0], l1b[0])
    q = jnp.dot(t, wq[0], preferred_element_type=F32) + bq[0]
    kk = jnp.dot(t, wk[0], preferred_element_type=F32) + bk[0]
    v = jnp.dot(t, wv[0], preferred_element_type=F32) + bv[0]
    ohs = []
    for h in range(HEADS):
        qh = q[:, h * hd:(h + 1) * hd]
        kh = kk[:, h * hd:(h + 1) * hd]
        vh = v[:, h * hd:(h + 1) * hd]
        sc = lax.dot_general(qh, kh, (((1,), (1,)), ((), ())),
                             preferred_element_type=F32) * (1.0 / 4.0)
        sc = jnp.where(colmask, sc, -1e30)
        sc = sc - sc.max(-1, keepdims=True)
        ex = jnp.exp(sc)
        att = ex / ex.sum(-1, keepdims=True)
        ohs.append(jnp.dot(att, vh, preferred_element_type=F32))
    o = jnp.concatenate(ohs, axis=1)
    ynew = y[...] + jnp.dot(o, wo[0], preferred_element_type=F32) + bo[0]
    t2 = _ln(ynew, l2s[0], l2b[0])
    f = jnp.dot(jax.nn.gelu(jnp.dot(t2, wf1[0], preferred_element_type=F32)
                            + bf1[0]),
                wf2[0], preferred_element_type=F32) + bf2[0]
    y[...] = ynew + f

    @pl.when(i == DEPTH - 1)
    def _():
        out_ref[...] = (jnp.dot(y[0:1, :], wcls[...],
                                preferred_element_type=F32) + bcls[...])


@jax.jit
def _transformer(y0, p):
    cube = lambda i: (i, 0, 0)
    fixed = lambda i: (0, 0)
    row3 = lambda n: pl.BlockSpec((1, 1, n), cube)
    r2 = lambda a: a[:, None, :]
    return pl.pallas_call(
        _tf_body,
        grid=(DEPTH,),
        in_specs=[
            pl.BlockSpec((SPAD, D), fixed),
            row3(D), row3(D),
            pl.BlockSpec((1, D, D), cube), row3(D),
            pl.BlockSpec((1, D, D), cube), row3(D),
            pl.BlockSpec((1, D, D), cube), row3(D),
            pl.BlockSpec((1, D, D), cube), row3(D),
            row3(D), row3(D),
            pl.BlockSpec((1, D, 4 * D), cube), row3(4 * D),
            pl.BlockSpec((1, 4 * D, D), cube), row3(D),
            pl.BlockSpec((D, NUM_CLASSES), fixed),
            pl.BlockSpec((1, NUM_CLASSES), fixed),
        ],
        out_specs=pl.BlockSpec((1, NUM_CLASSES), fixed),
        out_shape=jax.ShapeDtypeStruct((1, NUM_CLASSES), F32),
        scratch_shapes=[pltpu.VMEM((SPAD, D), F32)],
    )(y0, r2(p['ln1_s']), r2(p['ln1_b']), p['Wq'], r2(p['bq']),
      p['Wk'], r2(p['bk']), p['Wv'], r2(p['bv']), p['Wo'], r2(p['bo']),
      r2(p['ln2_s']), r2(p['ln2_b']),
      p['Wf1'], r2(p['bf1']), p['Wf2'], r2(p['bf2']),
      p['W_cls'], p['b_cls'][None, :])


def kernel(x, params, edge_index, cluster_indices, cluster_mask):
    p = params
    xp = jnp.pad(x, ((0, NPAD - N), (0, 0)))
    sl = jnp.arange(N, dtype=jnp.int32)
    npad_e = EPAD - E - N
    src = jnp.concatenate([edge_index[0].astype(jnp.int32), sl,
                           jnp.full((npad_e,), DUMMY, jnp.int32)])
    dst = jnp.concatenate([edge_index[1].astype(jnp.int32), sl,
                           jnp.full((npad_e,), DUMMY, jnp.int32)])
    src2 = src.reshape(32, 44, 128)
    dst2 = dst.reshape(32, 44, 128)
    iota_rows = jnp.arange(NROWS, dtype=jnp.int32).reshape(1, NROWS)

    halves = lambda z: jnp.stack([z[:, 32 * q:32 * q + 32] for q in range(4)])
    z1, s1, d1 = _proj1(xp, p['W_in'], p['b_in'], p['W1'],
                        p['a_src1'], p['a_dst1'])
    pp1 = _edge_phase(halves(z1), s1.reshape(-1), d1.reshape(-1),
                      src2, dst2, iota_rows)
    z2, s2, d2 = _proj2(pp1, p['b1'], p['W2'], p['a_src2'], p['a_dst2'])
    pp2 = _edge_phase(halves(z2), s2.reshape(-1), d2.reshape(-1),
                      src2, dst2, iota_rows)
    h2 = _sum2(pp2, p['b2'])

    cit = jnp.pad(cluster_indices.T.astype(jnp.int32), ((0, 0), (0, 7)))
    idx = cit.reshape(32, 5, 64)
    mt = jnp.pad(cluster_mask.T, ((0, 0), (0, 7)))
    mbc = jnp.broadcast_to(mt[:, :, None], (K, SPAD, D))

    g = _cluster_gather(h2, idx)
    ce = _emb(g.reshape(K, SPAD, D), mbc, p['W_emb'].reshape(K, D, D),
              p['b_emb'])
    y0 = jnp.concatenate([p['class_token'][None, :], ce[:C],
                          jnp.zeros((SPAD - C - 1, D), F32)], axis=0)
    return _transformer(y0, p)


# speedup vs baseline: 7.6700x; 1.0003x over previous
import dataclasses
import functools

import jax
import jax.numpy as jnp
from jax import lax
from jax.experimental import pallas as pl
from jax.experimental.pallas import tpu as pltpu
from jax.experimental.pallas import tpu_sc as plsc

N = 10000
E = 160000
D = 128
K = 32
C = 313
HEADS = 8
DEPTH = 8
NUM_CLASSES = 10

NPAD = 10240
EPAD = 180224
EW32 = EPAD // 32
EW16 = EPAD // 16
NROWS = NPAD // 128
DUMMY = N + 64
SPAD = 320
F32 = jnp.float32

_mesh = plsc.VectorSubcoreMesh(core_axis_name="c", subcore_axis_name="s")

_sc_params = pltpu.CompilerParams(use_tc_tiling_on_sc=False)
if "needs_layout_passes" in pltpu.CompilerParams.__dataclass_fields__:
    _sc_params = dataclasses.replace(_sc_params, needs_layout_passes=False)


def _edge_body(z_hbm, s_hbm, d_hbm, src2_hbm, dst2_hbm,
               iota_hbm, out_hbm,
               s_loc, d_loc, srcl_loc, dstl_loc, ebuf,
               den_loc, alpha_loc, gbuf0, gbuf1, sbuf0, sbuf1,
               mx_loc, mred_loc, iota_loc,
               out_sh, den_sh, mx_sh,
               gsem0, gsem1, ssem0, ssem1):
    cid = lax.axis_index("c")
    tid = lax.axis_index("s")

    scope = jax.named_scope
    pltpu.sync_copy(s_hbm, s_loc)
    pltpu.sync_copy(d_hbm, d_loc)
    pltpu.sync_copy(src2_hbm.at[2 * tid], srcl_loc.at[pl.ds(0, 44)])
    pltpu.sync_copy(src2_hbm.at[2 * tid + 1], srcl_loc.at[pl.ds(44, 44)])
    pltpu.sync_copy(dst2_hbm.at[2 * tid], dstl_loc.at[pl.ds(0, 44)])
    pltpu.sync_copy(dst2_hbm.at[2 * tid + 1], dstl_loc.at[pl.ds(44, 44)])
    pltpu.sync_copy(iota_hbm, iota_loc)

    zero16 = jnp.zeros((16,), F32)

    @pl.loop(0, NROWS)
    def _(r):
        for cc in range(8):
            den_loc[r, pl.ds(cc * 16, 16)] = zero16

    @pl.when(tid == 0)
    def _():
        pltpu.sync_copy(den_loc, den_sh)

    sc1 = scope("p1"); sc1.__enter__()
    mx_loc[...] = jnp.full((16,), -3e38, F32)

    @pl.loop(0, 88)
    def _(r):
        for cc in range(8):
            si = srcl_loc[r, pl.ds(cc * 16, 16)]
            di = dstl_loc[r, pl.ds(cc * 16, 16)]
            ev = plsc.load_gather(s_loc, [si]) + plsc.load_gather(d_loc, [di])
            ev = jnp.where(ev > 0, ev, 0.2 * ev)
            ebuf[r, pl.ds(cc * 16, 16)] = ev
            mx_loc[...] = jnp.maximum(mx_loc[...], ev)

    sc1.__exit__(None, None, None)
    sc2 = scope("p2"); sc2.__enter__()
    pltpu.sync_copy(mx_loc, mx_sh.at[tid])
    plsc.subcore_barrier()
    pltpu.sync_copy(mx_sh, mred_loc)
    mv = mred_loc[0, :]
    for t in range(1, 16):
        mv = jnp.maximum(mv, mred_loc[t, :])
    mscal = jnp.max(mv)
    mvec = jnp.full((16,), mscal, F32)

    @pl.loop(0, 88)
    def _(r):
        for cc in range(8):
            di = dstl_loc[r, pl.ds(cc * 16, 16)]
            ex = jnp.exp(ebuf[r, pl.ds(cc * 16, 16)] - mvec)
            plsc.addupdate_scatter(den_loc, [di >> 7, di & 127], ex)

    pltpu.sync_copy(den_loc, den_sh.at[iota_loc.at[0]], add=True)
    plsc.subcore_barrier()
    pltpu.sync_copy(den_sh, den_loc)

    @pl.loop(0, 44)
    def _(r):
        for cc in range(8):
            di = dstl_loc[cid * 44 + r, pl.ds(cc * 16, 16)]
            ex = jnp.exp(ebuf[cid * 44 + r, pl.ds(cc * 16, 16)] - mvec)
            dn = plsc.load_gather(den_loc, [di >> 7, di & 127])
            alpha_loc[pl.ds(r * 128 + cc * 16, 16)] = ex / (dn + 1e-16)

    sc2.__exit__(None, None, None)
    gbufs = (gbuf0, gbuf1)
    sbufs = (sbuf0, sbuf1)
    gsems = (gsem0, gsem1)
    ssems = (ssem0, ssem1)
    for h in range(4):
        scz = scope("hzero"); scz.__enter__()
        @pl.loop(0, 128)
        def _(r):
            for cc in range(2):
                gbuf0[r, pl.ds(cc * 16, 16)] = zero16
        for c5 in range(5):
            pltpu.sync_copy(gbuf0, out_sh.at[pl.ds(tid * 640 + c5 * 128, 128)])
        plsc.subcore_barrier()

        scz.__exit__(None, None, None)
        scc = scope("hchunks"); scc.__enter__()
        for b in range(2):
            pltpu.async_copy(z_hbm.at[h].at[srcl_loc.at[cid * 44 + b]],
                             gbufs[b], gsems[b])

        @pl.loop(0, 44, step=2)
        def _(jj):
            for b in range(2):
                j = jj + b
                pltpu.make_async_copy(
                    z_hbm.at[h].at[srcl_loc.at[cid * 44 + j]],
                    gbufs[b], gsems[b]).wait()

                @pl.when(jj > 0)
                def _():
                    pltpu.make_async_copy(
                        sbufs[b], out_sh.at[dstl_loc.at[cid * 44 + j]],
                        ssems[b]).wait()

                @pl.loop(0, 128, step=16)
                def _(r):
                    av16 = alpha_loc[pl.ds(j * 128 + r, 16)]
                    for k in range(16):
                        avk = lax.gather(
                            av16, jnp.full((16, 1), k, jnp.int32),
                            lax.GatherDimensionNumbers(
                                offset_dims=(), collapsed_slice_dims=(0,),
                                start_index_map=(0,)),
                            (1,), indices_are_sorted=True, unique_indices=False,
                            mode=lax.GatherScatterMode.PROMISE_IN_BOUNDS)
                        for cc in range(2):
                            sbufs[b][r + k, pl.ds(cc * 16, 16)] = (
                                gbufs[b][r + k, pl.ds(cc * 16, 16)] * avk)

                pltpu.async_copy(sbufs[b],
                                 out_sh.at[dstl_loc.at[cid * 44 + j]],
                                 ssems[b], add=True)

                @pl.when(j + 2 < 44)
                def _():
                    pltpu.async_copy(
                        z_hbm.at[h].at[srcl_loc.at[cid * 44 + j + 2]],
                        gbufs[b], gsems[b])

        for b in range(2):
            pltpu.make_async_copy(sbufs[b],
                                  out_sh.at[dstl_loc.at[cid * 44]],
                                  ssems[b]).wait()

        scc.__exit__(None, None, None)
        scw = scope("hwout"); scw.__enter__()
        plsc.subcore_barrier()
        for c5 in range(5):
            rows = pl.ds(tid * 640 + c5 * 128, 128)
            pltpu.sync_copy(out_sh.at[rows], out_hbm.at[cid].at[h].at[rows])
        scw.__exit__(None, None, None)


@jax.jit
def _edge_phase(z, s, d, src2, dst2, iota_rows):
    kfn = pl.kernel(
        _edge_body,
        out_type=jax.ShapeDtypeStruct((2, 4, NPAD, 32), F32),
        mesh=_mesh,
        compiler_params=_sc_params,
        scratch_types=[
            pltpu.VMEM((NPAD,), F32),
            pltpu.VMEM((NPAD,), F32),
            pltpu.VMEM((88, 128), jnp.int32),
            pltpu.VMEM((88, 128), jnp.int32),
            pltpu.VMEM((88, 128), F32),
            pltpu.VMEM((NROWS, 128), F32),
            pltpu.VMEM((EW32,), F32),
            pltpu.VMEM((128, 32), F32),
            pltpu.VMEM((128, 32), F32),
            pltpu.VMEM((128, 32), F32),
            pltpu.VMEM((128, 32), F32),
            pltpu.VMEM((16,), F32),
            pltpu.VMEM((16, 16), F32),
            pltpu.VMEM((1, NROWS), jnp.int32),
            pltpu.VMEM_SHARED((NPAD, 32), F32),
            pltpu.VMEM_SHARED((NROWS, 128), F32),
            pltpu.VMEM_SHARED((16, 16), F32),
            pltpu.SemaphoreType.DMA,
            pltpu.SemaphoreType.DMA,
            pltpu.SemaphoreType.DMA,
            pltpu.SemaphoreType.DMA,
        ],
    )
    return kfn(z, s, d, src2, dst2, iota_rows)


def _gather_body(h_hbm, idx_hbm, g_hbm, idx_loc, rowbuf):
    cid = lax.axis_index("c")
    tid = lax.axis_index("s")
    wid = tid * 2 + cid
    pltpu.sync_copy(idx_hbm.at[wid], idx_loc)
    for c in range(5):
        pltpu.sync_copy(h_hbm.at[idx_loc.at[c]], rowbuf)
        pltpu.sync_copy(rowbuf, g_hbm.at[pl.ds(wid * 320 + c * 64, 64)])


@jax.jit
def _cluster_gather(h2, idx):
    kfn = pl.kernel(
        _gather_body,
        out_type=jax.ShapeDtypeStruct((NPAD, D), F32),
        mesh=_mesh,
        scratch_types=[
            pltpu.VMEM((5, 64), jnp.int32),
            pltpu.VMEM((64, 128), F32),
        ],
    )
    return kfn(h2, idx)


_ROWS_BLK = 256
_N_BLKS = NPAD // _ROWS_BLK


def _proj1_body(x_ref, wi_ref, bi_ref, w_ref, asrc_ref, adst_ref,
                z_ref, s_ref, d_ref):
    h = jnp.dot(x_ref[...], wi_ref[...], preferred_element_type=F32) + bi_ref[...]
    z = jnp.dot(h, w_ref[...], preferred_element_type=F32)
    z_ref[...] = z
    s_ref[...] = jnp.dot(z, asrc_ref[...], preferred_element_type=F32)
    d_ref[...] = jnp.dot(z, adst_ref[...], preferred_element_type=F32)


@jax.jit
def _proj1(x, w_in, b_in, w, a_src, a_dst):
    full = lambda i: (0, 0)
    return pl.pallas_call(
        _proj1_body,
        grid=(_N_BLKS,),
        in_specs=[
            pl.BlockSpec((_ROWS_BLK, D), lambda i: (i, 0)),
            pl.BlockSpec((D, D), full),
            pl.BlockSpec((1, D), full),
            pl.BlockSpec((D, D), full),
            pl.BlockSpec((D, 1), full),
            pl.BlockSpec((D, 1), full),
        ],
        out_specs=[
            pl.BlockSpec((_ROWS_BLK, D), lambda i: (i, 0)),
            pl.BlockSpec((_ROWS_BLK, 1), lambda i: (i, 0)),
            pl.BlockSpec((_ROWS_BLK, 1), lambda i: (i, 0)),
        ],
        out_shape=[
            jax.ShapeDtypeStruct((NPAD, D), F32),
            jax.ShapeDtypeStruct((NPAD, 1), F32),
            jax.ShapeDtypeStruct((NPAD, 1), F32),
        ],
    )(x, w_in, b_in[None, :], w, a_src[:, None], a_dst[:, None])


def _proj2_body(p_ref, b_ref, w_ref, asrc_ref, adst_ref, z_ref, s_ref, d_ref):
    h = jnp.concatenate([p_ref[0, q] + p_ref[1, q] for q in range(4)],
                        axis=-1) + b_ref[...]
    z = jnp.dot(h, w_ref[...], preferred_element_type=F32)
    z_ref[...] = z
    s_ref[...] = jnp.dot(z, asrc_ref[...], preferred_element_type=F32)
    d_ref[...] = jnp.dot(z, adst_ref[...], preferred_element_type=F32)


@jax.jit
def _proj2(p, b, w, a_src, a_dst):
    full = lambda i: (0, 0)
    return pl.pallas_call(
        _proj2_body,
        grid=(_N_BLKS,),
        in_specs=[
            pl.BlockSpec((2, 4, _ROWS_BLK, 32), lambda i: (0, 0, i, 0)),
            pl.BlockSpec((1, D), full),
            pl.BlockSpec((D, D), full),
            pl.BlockSpec((D, 1), full),
            pl.BlockSpec((D, 1), full),
        ],
        out_specs=[
            pl.BlockSpec((_ROWS_BLK, D), lambda i: (i, 0)),
            pl.BlockSpec((_ROWS_BLK, 1), lambda i: (i, 0)),
            pl.BlockSpec((_ROWS_BLK, 1), lambda i: (i, 0)),
        ],
        out_shape=[
            jax.ShapeDtypeStruct((NPAD, D), F32),
            jax.ShapeDtypeStruct((NPAD, 1), F32),
            jax.ShapeDtypeStruct((NPAD, 1), F32),
        ],
    )(p, b[None, :], w, a_src[:, None], a_dst[:, None])


def _sum2_body(p_ref, b_ref, h_ref):
    h_ref[...] = jnp.concatenate([p_ref[0, q] + p_ref[1, q]
                                  for q in range(4)], axis=-1) + b_ref[...]


@jax.jit
def _sum2(p, b):
    return pl.pallas_call(
        _sum2_body,
        grid=(_N_BLKS,),
        in_specs=[
            pl.BlockSpec((2, 4, _ROWS_BLK, 32), lambda i: (0, 0, i, 0)),
            pl.BlockSpec((1, D), lambda i: (0, 0)),
        ],
        out_specs=pl.BlockSpec((_ROWS_BLK, D), lambda i: (i, 0)),
        out_shape=jax.ShapeDtypeStruct((NPAD, D), F32),
    )(p, b[None, :])


def _emb_body(g_ref, m_ref, w_ref, b_ref, ce_ref, acc):
    k = pl.program_id(0)

    @pl.when(k == 0)
    def _():
        acc[...] = jnp.zeros_like(acc)

    acc[...] += jnp.dot(g_ref[0] * m_ref[0], w_ref[0],
                        preferred_element_type=F32)

    @pl.when(k == K - 1)
    def _():
        ce_ref[...] = acc[...] + b_ref[...]


@jax.jit
def _emb(g, m, w, b):
    return pl.pallas_call(
        _emb_body,
        grid=(K,),
        in_specs=[
            pl.BlockSpec((1, SPAD, D), lambda k: (k, 0, 0)),
            pl.BlockSpec((1, SPAD, D), lambda k: (k, 0, 0)),
            pl.BlockSpec((1, D, D), lambda k: (k, 0, 0)),
            pl.BlockSpec((1, D), lambda k: (0, 0)),
        ],
        out_specs=pl.BlockSpec((SPAD, D), lambda k: (0, 0)),
        out_shape=jax.ShapeDtypeStruct((SPAD, D), F32),
        scratch_shapes=[pltpu.VMEM((SPAD, D), F32)],
    )(g, m, w, b[None, :])


def _ln(x, s, b):
    m = x.mean(-1, keepdims=True)
    v = ((x - m) ** 2).mean(-1, keepdims=True)
    return (x - m) / jnp.sqrt(v + 1e-5) * s + b


def _tf_body(y0_ref, l1s, l1b, wq, bq, wk, bk, wv, bv, wo, bo,
             l2s, l2b, wf1, bf1, wf2, bf2, wcls, bcls, out_ref, y):
    i = pl.program_id(0)

    @pl.when(i == 0)
    def _():
        y[...] = y0_ref[...]

    hd = D // HEADS
    colmask = lax.broadcasted_iota(jnp.int32, (SPAD, SPAD), 1) < (C + 1)

    t = _ln(y[...], l1s[0], l1b[0])
    q = jnp.dot(t, wq[0], preferred_element_type=F32) + bq[0]
    kk = jnp.dot(t, wk[0], preferred_element_type=F32) + bk[0]
    v = jnp.dot(t, wv[0], preferred_element_type=F32) + bv[0]
    ohs = []
    for h in range(HEADS):
        qh = q[:, h * hd:(h + 1) * hd]
        kh = kk[:, h * hd:(h + 1) * hd]
        vh = v[:, h * hd:(h + 1) * hd]
        sc = lax.dot_general(qh, kh, (((1,), (1,)), ((), ())),
                             preferred_element_type=F32) * (1.0 / 4.0)
        sc = jnp.where(colmask, sc, -1e30)
        sc = sc - sc.max(-1, keepdims=True)
        ex = jnp.exp(sc)
        att = ex / ex.sum(-1, keepdims=True)
        ohs.append(jnp.dot(att, vh, preferred_element_type=F32))
    o = jnp.concatenate(ohs, axis=1)
    ynew = y[...] + jnp.dot(o, wo[0], preferred_element_type=F32) + bo[0]
    t2 = _ln(ynew, l2s[0], l2b[0])
    f = jnp.dot(jax.nn.gelu(jnp.dot(t2, wf1[0], preferred_element_type=F32)
                            + bf1[0]),
                wf2[0], preferred_element_type=F32) + bf2[0]
    y[...] = ynew + f

    @pl.when(i == DEPTH - 1)
    def _():
        out_ref[...] = (jnp.dot(y[0:1, :], wcls[...],
                                preferred_element_type=F32) + bcls[...])


@jax.jit
def _transformer(y0, p):
    cube = lambda i: (i, 0, 0)
    fixed = lambda i: (0, 0)
    row3 = lambda n: pl.BlockSpec((1, 1, n), cube)
    r2 = lambda a: a[:, None, :]
    return pl.pallas_call(
        _tf_body,
        grid=(DEPTH,),
        in_specs=[
            pl.BlockSpec((SPAD, D), fixed),
            row3(D), row3(D),
            pl.BlockSpec((1, D, D), cube), row3(D),
            pl.BlockSpec((1, D, D), cube), row3(D),
            pl.BlockSpec((1, D, D), cube), row3(D),
            pl.BlockSpec((1, D, D), cube), row3(D),
            row3(D), row3(D),
            pl.BlockSpec((1, D, 4 * D), cube), row3(4 * D),
            pl.BlockSpec((1, 4 * D, D), cube), row3(D),
            pl.BlockSpec((D, NUM_CLASSES), fixed),
            pl.BlockSpec((1, NUM_CLASSES), fixed),
        ],
        out_specs=pl.BlockSpec((1, NUM_CLASSES), fixed),
        out_shape=jax.ShapeDtypeStruct((1, NUM_CLASSES), F32),
        scratch_shapes=[pltpu.VMEM((SPAD, D), F32)],
    )(y0, r2(p['ln1_s']), r2(p['ln1_b']), p['Wq'], r2(p['bq']),
      p['Wk'], r2(p['bk']), p['Wv'], r2(p['bv']), p['Wo'], r2(p['bo']),
      r2(p['ln2_s']), r2(p['ln2_b']),
      p['Wf1'], r2(p['bf1']), p['Wf2'], r2(p['bf2']),
      p['W_cls'], p['b_cls'][None, :])


def kernel(x, params, edge_index, cluster_indices, cluster_mask):
    p = params
    xp = jnp.pad(x, ((0, NPAD - N), (0, 0)))
    sl = jnp.arange(N, dtype=jnp.int32)
    npad_e = EPAD - E - N
    src = jnp.concatenate([edge_index[0].astype(jnp.int32), sl,
                           jnp.full((npad_e,), DUMMY, jnp.int32)])
    dst = jnp.concatenate([edge_index[1].astype(jnp.int32), sl,
                           jnp.full((npad_e,), DUMMY, jnp.int32)])
    src2 = src.reshape(32, 44, 128)
    dst2 = dst.reshape(32, 44, 128)
    iota_rows = jnp.arange(NROWS, dtype=jnp.int32).reshape(1, NROWS)

    halves = lambda z: jnp.stack([z[:, 32 * q:32 * q + 32] for q in range(4)])
    z1, s1, d1 = _proj1(xp, p['W_in'], p['b_in'], p['W1'],
                        p['a_src1'], p['a_dst1'])
    pp1 = _edge_phase(halves(z1), s1.reshape(-1), d1.reshape(-1),
                      src2, dst2, iota_rows)
    z2, s2, d2 = _proj2(pp1, p['b1'], p['W2'], p['a_src2'], p['a_dst2'])
    pp2 = _edge_phase(halves(z2), s2.reshape(-1), d2.reshape(-1),
                      src2, dst2, iota_rows)
    h2 = _sum2(pp2, p['b2'])

    cit = jnp.pad(cluster_indices.T.astype(jnp.int32), ((0, 0), (0, 7)))
    idx = cit.reshape(32, 5, 64)
    mt = jnp.pad(cluster_mask.T, ((0, 0), (0, 7)))
    mbc = jnp.broadcast_to(mt[:, :, None], (K, SPAD, D))

    g = _cluster_gather(h2, idx)
    ce = _emb(g.reshape(K, SPAD, D), mbc, p['W_emb'].reshape(K, D, D),
              p['b_emb'])
    y0 = jnp.concatenate([p['class_token'][None, :], ce[:C],
                          jnp.zeros((SPAD - C - 1, D), F32)], axis=0)
    return _transformer(y0, p)


# speedup vs baseline: 15.1510x; 1.9754x over previous
import dataclasses
import functools

import jax
import jax.numpy as jnp
from jax import lax
from jax.experimental import pallas as pl
from jax.experimental.pallas import tpu as pltpu
from jax.experimental.pallas import tpu_sc as plsc

N = 10000
E = 160000
D = 128
K = 32
C = 313
HEADS = 8
DEPTH = 8
NUM_CLASSES = 10

NPAD = 10240
EPAD = 180224
EW32 = EPAD // 32
EW16 = EPAD // 16
NROWS = NPAD // 128
DUMMY = N + 64
SPAD = 320
F32 = jnp.float32

_mesh = plsc.VectorSubcoreMesh(core_axis_name="c", subcore_axis_name="s")

_sc_params = pltpu.CompilerParams(use_tc_tiling_on_sc=False)
if "needs_layout_passes" in pltpu.CompilerParams.__dataclass_fields__:
    _sc_params = dataclasses.replace(_sc_params, needs_layout_passes=False)


def _edge_body(z_hbm, s_hbm, d_hbm, src2_hbm, dst2_hbm,
               iota_hbm, out_hbm,
               s_loc, d_loc, srcl_loc, dstl_loc, ebuf,
               den_loc, alpha_loc, gbuf0, gbuf1, sbuf0, sbuf1,
               mx_loc, mred_loc, iota_loc,
               out_sh, den_sh, mx_sh,
               gsem0, gsem1, ssem0, ssem1):
    cid = lax.axis_index("c")
    tid = lax.axis_index("s")

    scope = jax.named_scope
    pltpu.sync_copy(s_hbm, s_loc)
    pltpu.sync_copy(d_hbm, d_loc)
    pltpu.sync_copy(src2_hbm.at[2 * tid], srcl_loc.at[pl.ds(0, 44)])
    pltpu.sync_copy(src2_hbm.at[2 * tid + 1], srcl_loc.at[pl.ds(44, 44)])
    pltpu.sync_copy(dst2_hbm.at[2 * tid], dstl_loc.at[pl.ds(0, 44)])
    pltpu.sync_copy(dst2_hbm.at[2 * tid + 1], dstl_loc.at[pl.ds(44, 44)])
    pltpu.sync_copy(iota_hbm, iota_loc)

    zero16 = jnp.zeros((16,), F32)

    @pl.loop(0, NROWS)
    def _(r):
        for cc in range(8):
            den_loc[r, pl.ds(cc * 16, 16)] = zero16

    @pl.when(tid == 0)
    def _():
        pltpu.sync_copy(den_loc, den_sh)

    sc1 = scope("p1"); sc1.__enter__()
    mx_loc[...] = jnp.full((16,), -3e38, F32)

    @pl.loop(0, 88)
    def _(r):
        for cc in range(8):
            si = srcl_loc[r, pl.ds(cc * 16, 16)]
            di = dstl_loc[r, pl.ds(cc * 16, 16)]
            ev = plsc.load_gather(s_loc, [si]) + plsc.load_gather(d_loc, [di])
            ev = jnp.where(ev > 0, ev, 0.2 * ev)
            ebuf[r, pl.ds(cc * 16, 16)] = ev
            mx_loc[...] = jnp.maximum(mx_loc[...], ev)

    sc1.__exit__(None, None, None)
    sc2 = scope("p2"); sc2.__enter__()
    pltpu.sync_copy(mx_loc, mx_sh.at[tid])
    plsc.subcore_barrier()
    pltpu.sync_copy(mx_sh, mred_loc)
    mv = mred_loc[0, :]
    for t in range(1, 16):
        mv = jnp.maximum(mv, mred_loc[t, :])
    mscal = jnp.max(mv)
    mvec = jnp.full((16,), mscal, F32)

    @pl.loop(0, 88)
    def _(r):
        for cc in range(8):
            di = dstl_loc[r, pl.ds(cc * 16, 16)]
            ex = jnp.exp(ebuf[r, pl.ds(cc * 16, 16)] - mvec)
            plsc.addupdate_scatter(den_loc, [di >> 7, di & 127], ex)

    pltpu.sync_copy(den_loc, den_sh.at[iota_loc.at[0]], add=True)
    plsc.subcore_barrier()
    pltpu.sync_copy(den_sh, den_loc)

    @pl.loop(0, 44)
    def _(r):
        for cc in range(8):
            di = dstl_loc[cid * 44 + r, pl.ds(cc * 16, 16)]
            ex = jnp.exp(ebuf[cid * 44 + r, pl.ds(cc * 16, 16)] - mvec)
            dn = plsc.load_gather(den_loc, [di >> 7, di & 127])
            alpha_loc[pl.ds(r * 128 + cc * 16, 16)] = ex / (dn + 1e-16)

    sc2.__exit__(None, None, None)
    gbufs = (gbuf0, gbuf1)
    sbufs = (sbuf0, sbuf1)
    gsems = (gsem0, gsem1)
    ssems = (ssem0, ssem1)
    for h in range(4):
        scz = scope("hzero"); scz.__enter__()
        @pl.loop(0, 128)
        def _(r):
            for cc in range(2):
                gbuf0[r, pl.ds(cc * 16, 16)] = zero16
        for c5 in range(5):
            pltpu.sync_copy(gbuf0, out_sh.at[pl.ds(tid * 640 + c5 * 128, 128)])
        plsc.subcore_barrier()

        scz.__exit__(None, None, None)
        scc = scope("hchunks"); scc.__enter__()
        for b in range(2):
            pltpu.async_copy(z_hbm.at[h].at[srcl_loc.at[cid * 44 + b]],
                             gbufs[b], gsems[b])

        @pl.loop(0, 44, step=2)
        def _(jj):
            for b in range(2):
                j = jj + b
                pltpu.make_async_copy(
                    z_hbm.at[h].at[srcl_loc.at[cid * 44 + j]],
                    gbufs[b], gsems[b]).wait()

                @pl.when(jj > 0)
                def _():
                    pltpu.make_async_copy(
                        sbufs[b], out_sh.at[dstl_loc.at[cid * 44 + j]],
                        ssems[b]).wait()

                @pl.loop(0, 128, step=16)
                def _(r):
                    av16 = alpha_loc[pl.ds(j * 128 + r, 16)]
                    for k in range(16):
                        avk = lax.gather(
                            av16, jnp.full((16, 1), k, jnp.int32),
                            lax.GatherDimensionNumbers(
                                offset_dims=(), collapsed_slice_dims=(0,),
                                start_index_map=(0,)),
                            (1,), indices_are_sorted=True, unique_indices=False,
                            mode=lax.GatherScatterMode.PROMISE_IN_BOUNDS)
                        for cc in range(2):
                            sbufs[b][r + k, pl.ds(cc * 16, 16)] = (
                                gbufs[b][r + k, pl.ds(cc * 16, 16)] * avk)

                pltpu.async_copy(sbufs[b],
                                 out_sh.at[dstl_loc.at[cid * 44 + j]],
                                 ssems[b], add=True)

                @pl.when(j + 2 < 44)
                def _():
                    pltpu.async_copy(
                        z_hbm.at[h].at[srcl_loc.at[cid * 44 + j + 2]],
                        gbufs[b], gsems[b])

        for b in range(2):
            pltpu.make_async_copy(sbufs[b],
                                  out_sh.at[dstl_loc.at[cid * 44]],
                                  ssems[b]).wait()

        scc.__exit__(None, None, None)
        scw = scope("hwout"); scw.__enter__()
        plsc.subcore_barrier()
        for c5 in range(5):
            rows = pl.ds(tid * 640 + c5 * 128, 128)
            pltpu.sync_copy(out_sh.at[rows], out_hbm.at[cid].at[h].at[rows])
        scw.__exit__(None, None, None)


@jax.jit
def _edge_phase(z, s, d, src2, dst2, iota_rows):
    kfn = pl.kernel(
        _edge_body,
        out_type=jax.ShapeDtypeStruct((2, 4, NPAD, 32), F32),
        mesh=_mesh,
        compiler_params=_sc_params,
        scratch_types=[
            pltpu.VMEM((NPAD,), F32),
            pltpu.VMEM((NPAD,), F32),
            pltpu.VMEM((88, 128), jnp.int32),
            pltpu.VMEM((88, 128), jnp.int32),
            pltpu.VMEM((88, 128), F32),
            pltpu.VMEM((NROWS, 128), F32),
            pltpu.VMEM((EW32,), F32),
            pltpu.VMEM((128, 32), F32),
            pltpu.VMEM((128, 32), F32),
            pltpu.VMEM((128, 32), F32),
            pltpu.VMEM((128, 32), F32),
            pltpu.VMEM((16,), F32),
            pltpu.VMEM((16, 16), F32),
            pltpu.VMEM((1, NROWS), jnp.int32),
            pltpu.VMEM_SHARED((NPAD, 32), F32),
            pltpu.VMEM_SHARED((NROWS, 128), F32),
            pltpu.VMEM_SHARED((16, 16), F32),
            pltpu.SemaphoreType.DMA,
            pltpu.SemaphoreType.DMA,
            pltpu.SemaphoreType.DMA,
            pltpu.SemaphoreType.DMA,
        ],
    )
    return kfn(z, s, d, src2, dst2, iota_rows)


def _gather_body(h_hbm, idx_hbm, g_hbm, idx_loc, rowbuf):
    cid = lax.axis_index("c")
    tid = lax.axis_index("s")
    wid = tid * 2 + cid
    pltpu.sync_copy(idx_hbm.at[wid], idx_loc)
    for c in range(5):
        pltpu.sync_copy(h_hbm.at[idx_loc.at[c]], rowbuf)
        pltpu.sync_copy(rowbuf, g_hbm.at[pl.ds(wid * 320 + c * 64, 64)])


@jax.jit
def _cluster_gather(h2, idx):
    kfn = pl.kernel(
        _gather_body,
        out_type=jax.ShapeDtypeStruct((NPAD, D), F32),
        mesh=_mesh,
        scratch_types=[
            pltpu.VMEM((5, 64), jnp.int32),
            pltpu.VMEM((64, 128), F32),
        ],
    )
    return kfn(h2, idx)


_ROWS_BLK = 256
_N_BLKS = NPAD // _ROWS_BLK


def _proj1_body(x_ref, wi_ref, bi_ref, w_ref, asrc_ref, adst_ref,
                z_ref, s_ref, d_ref):
    h = jnp.dot(x_ref[...], wi_ref[...], preferred_element_type=F32) + bi_ref[...]
    z = jnp.dot(h, w_ref[...], preferred_element_type=F32)
    z_ref[...] = z
    s_ref[...] = jnp.dot(z, asrc_ref[...], preferred_element_type=F32)
    d_ref[...] = jnp.dot(z, adst_ref[...], preferred_element_type=F32)


@jax.jit
def _proj1(x, w_in, b_in, w, a_src, a_dst):
    full = lambda i: (0, 0)
    return pl.pallas_call(
        _proj1_body,
        grid=(_N_BLKS,),
        in_specs=[
            pl.BlockSpec((_ROWS_BLK, D), lambda i: (i, 0)),
            pl.BlockSpec((D, D), full),
            pl.BlockSpec((1, D), full),
            pl.BlockSpec((D, D), full),
            pl.BlockSpec((D, 1), full),
            pl.BlockSpec((D, 1), full),
        ],
        out_specs=[
            pl.BlockSpec((_ROWS_BLK, D), lambda i: (i, 0)),
            pl.BlockSpec((_ROWS_BLK, 1), lambda i: (i, 0)),
            pl.BlockSpec((_ROWS_BLK, 1), lambda i: (i, 0)),
        ],
        out_shape=[
            jax.ShapeDtypeStruct((NPAD, D), F32),
            jax.ShapeDtypeStruct((NPAD, 1), F32),
            jax.ShapeDtypeStruct((NPAD, 1), F32),
        ],
    )(x, w_in, b_in[None, :], w, a_src[:, None], a_dst[:, None])


def _proj2_body(p_ref, b_ref, w_ref, asrc_ref, adst_ref, z_ref, s_ref, d_ref):
    h = jnp.concatenate([p_ref[0, q] + p_ref[1, q] for q in range(4)],
                        axis=-1) + b_ref[...]
    z = jnp.dot(h, w_ref[...], preferred_element_type=F32)
    z_ref[...] = z
    s_ref[...] = jnp.dot(z, asrc_ref[...], preferred_element_type=F32)
    d_ref[...] = jnp.dot(z, adst_ref[...], preferred_element_type=F32)


@jax.jit
def _proj2(p, b, w, a_src, a_dst):
    full = lambda i: (0, 0)
    return pl.pallas_call(
        _proj2_body,
        grid=(_N_BLKS,),
        in_specs=[
            pl.BlockSpec((2, 4, _ROWS_BLK, 32), lambda i: (0, 0, i, 0)),
            pl.BlockSpec((1, D), full),
            pl.BlockSpec((D, D), full),
            pl.BlockSpec((D, 1), full),
            pl.BlockSpec((D, 1), full),
        ],
        out_specs=[
            pl.BlockSpec((_ROWS_BLK, D), lambda i: (i, 0)),
            pl.BlockSpec((_ROWS_BLK, 1), lambda i: (i, 0)),
            pl.BlockSpec((_ROWS_BLK, 1), lambda i: (i, 0)),
        ],
        out_shape=[
            jax.ShapeDtypeStruct((NPAD, D), F32),
            jax.ShapeDtypeStruct((NPAD, 1), F32),
            jax.ShapeDtypeStruct((NPAD, 1), F32),
        ],
    )(p, b[None, :], w, a_src[:, None], a_dst[:, None])


def _sum2_body(p_ref, b_ref, h_ref):
    h_ref[...] = jnp.concatenate([p_ref[0, q] + p_ref[1, q]
                                  for q in range(4)], axis=-1) + b_ref[...]


@jax.jit
def _sum2(p, b):
    return pl.pallas_call(
        _sum2_body,
        grid=(_N_BLKS,),
        in_specs=[
            pl.BlockSpec((2, 4, _ROWS_BLK, 32), lambda i: (0, 0, i, 0)),
            pl.BlockSpec((1, D), lambda i: (0, 0)),
        ],
        out_specs=pl.BlockSpec((_ROWS_BLK, D), lambda i: (i, 0)),
        out_shape=jax.ShapeDtypeStruct((NPAD, D), F32),
    )(p, b[None, :])


def _emb_body(g_ref, m_ref, w_ref, b_ref, ce_ref, acc):
    k = pl.program_id(0)

    @pl.when(k == 0)
    def _():
        acc[...] = jnp.zeros_like(acc)

    acc[...] += jnp.dot(g_ref[0] * m_ref[0], w_ref[0],
                        preferred_element_type=F32)

    @pl.when(k == K - 1)
    def _():
        ce_ref[...] = acc[...] + b_ref[...]


@jax.jit
def _emb(g, m, w, b):
    return pl.pallas_call(
        _emb_body,
        grid=(K,),
        in_specs=[
            pl.BlockSpec((1, SPAD, D), lambda k: (k, 0, 0)),
            pl.BlockSpec((1, SPAD, D), lambda k: (k, 0, 0)),
            pl.BlockSpec((1, D, D), lambda k: (k, 0, 0)),
            pl.BlockSpec((1, D), lambda k: (0, 0)),
        ],
        out_specs=pl.BlockSpec((SPAD, D), lambda k: (0, 0)),
        out_shape=jax.ShapeDtypeStruct((SPAD, D), F32),
        scratch_shapes=[pltpu.VMEM((SPAD, D), F32)],
    )(g, m, w, b[None, :])


def _ln(x, s, b):
    m = x.mean(-1, keepdims=True)
    v = ((x - m) ** 2).mean(-1, keepdims=True)
    return (x - m) / jnp.sqrt(v + 1e-5) * s + b


def _tf_body(y0_ref, l1s, l1b, wq, bq, wk, bk, wv, bv, wo, bo,
             l2s, l2b, wf1, bf1, wf2, bf2, wcls, bcls, out_ref, y):
    i = pl.program_id(0)

    @pl.when(i == 0)
    def _():
        y[...] = y0_ref[...]

    hd = D // HEADS
    colmask = lax.broadcasted_iota(jnp.int32, (SPAD, SPAD), 1) < (C + 1)

    t = _ln(y[...], l1s[0], l1b[0])
    q = jnp.dot(t, wq[0], preferred_element_type=F32) + bq[0]
    kk = jnp.dot(t, wk[0], preferred_element_type=F32) + bk[0]
    v = jnp.dot(t, wv[0], preferred_element_type=F32) + bv[0]
    ohs = []
    for h in range(HEADS):
        qh = q[:, h * hd:(h + 1) * hd]
        kh = kk[:, h * hd:(h + 1) * hd]
        vh = v[:, h * hd:(h + 1) * hd]
        sc = lax.dot_general(qh, kh, (((1,), (1,)), ((), ())),
                             preferred_element_type=F32) * (1.0 / 4.0)
        sc = jnp.where(colmask, sc, -1e30)
        sc = sc - sc.max(-1, keepdims=True)
        ex = jnp.exp(sc)
        att = ex / ex.sum(-1, keepdims=True)
        ohs.append(jnp.dot(att, vh, preferred_element_type=F32))
    o = jnp.concatenate(ohs, axis=1)
    ynew = y[...] + jnp.dot(o, wo[0], preferred_element_type=F32) + bo[0]
    t2 = _ln(ynew, l2s[0], l2b[0])
    f = jnp.dot(jax.nn.gelu(jnp.dot(t2, wf1[0], preferred_element_type=F32)
                            + bf1[0]),
                wf2[0], preferred_element_type=F32) + bf2[0]
    y[...] = ynew + f

    @pl.when(i == DEPTH - 1)
    def _():
        out_ref[...] = (jnp.dot(y[0:1, :], wcls[...],
                                preferred_element_type=F32) + bcls[...])


@jax.jit
def _transformer(y0, p):
    cube = lambda i: (i, 0, 0)
    fixed = lambda i: (0, 0)
    row3 = lambda n: pl.BlockSpec((1, 1, n), cube)
    r2 = lambda a: a[:, None, :]
    return pl.pallas_call(
        _tf_body,
        grid=(DEPTH,),
        in_specs=[
            pl.BlockSpec((SPAD, D), fixed),
            row3(D), row3(D),
            pl.BlockSpec((1, D, D), cube), row3(D),
            pl.BlockSpec((1, D, D), cube), row3(D),
            pl.BlockSpec((1, D, D), cube), row3(D),
            pl.BlockSpec((1, D, D), cube), row3(D),
            row3(D), row3(D),
            pl.BlockSpec((1, D, 4 * D), cube), row3(4 * D),
            pl.BlockSpec((1, 4 * D, D), cube), row3(D),
            pl.BlockSpec((D, NUM_CLASSES), fixed),
            pl.BlockSpec((1, NUM_CLASSES), fixed),
        ],
        out_specs=pl.BlockSpec((1, NUM_CLASSES), fixed),
        out_shape=jax.ShapeDtypeStruct((1, NUM_CLASSES), F32),
        scratch_shapes=[pltpu.VMEM((SPAD, D), F32)],
    )(y0, r2(p['ln1_s']), r2(p['ln1_b']), p['Wq'], r2(p['bq']),
      p['Wk'], r2(p['bk']), p['Wv'], r2(p['bv']), p['Wo'], r2(p['bo']),
      r2(p['ln2_s']), r2(p['ln2_b']),
      p['Wf1'], r2(p['bf1']), p['Wf2'], r2(p['bf2']),
      p['W_cls'], p['b_cls'][None, :])


def kernel(x, params, edge_index, cluster_indices, cluster_mask):
    p = params
    xp = jnp.pad(x, ((0, NPAD - N), (0, 0)))
    sl = jnp.arange(N, dtype=jnp.int32)
    npad_e = EPAD - E - N
    spread = N + (jnp.arange(npad_e, dtype=jnp.int32) % (NPAD - N))
    src = jnp.concatenate([edge_index[0].astype(jnp.int32), sl, spread])
    dst = jnp.concatenate([edge_index[1].astype(jnp.int32), sl, spread])
    src2 = src.reshape(32, 44, 128)
    dst2 = dst.reshape(32, 44, 128)
    iota_rows = jnp.arange(NROWS, dtype=jnp.int32).reshape(1, NROWS)

    halves = lambda z: jnp.stack([z[:, 32 * q:32 * q + 32] for q in range(4)])
    z1, s1, d1 = _proj1(xp, p['W_in'], p['b_in'], p['W1'],
                        p['a_src1'], p['a_dst1'])
    pp1 = _edge_phase(halves(z1), s1.reshape(-1), d1.reshape(-1),
                      src2, dst2, iota_rows)
    z2, s2, d2 = _proj2(pp1, p['b1'], p['W2'], p['a_src2'], p['a_dst2'])
    pp2 = _edge_phase(halves(z2), s2.reshape(-1), d2.reshape(-1),
                      src2, dst2, iota_rows)
    h2 = _sum2(pp2, p['b2'])

    cit = jnp.pad(cluster_indices.T.astype(jnp.int32), ((0, 0), (0, 7)))
    idx = cit.reshape(32, 5, 64)
    mt = jnp.pad(cluster_mask.T, ((0, 0), (0, 7)))
    mbc = jnp.broadcast_to(mt[:, :, None], (K, SPAD, D))

    g = _cluster_gather(h2, idx)
    ce = _emb(g.reshape(K, SPAD, D), mbc, p['W_emb'].reshape(K, D, D),
              p['b_emb'])
    y0 = jnp.concatenate([p['class_token'][None, :], ce[:C],
                          jnp.zeros((SPAD - C - 1, D), F32)], axis=0)
    return _transformer(y0, p)


# speedup vs baseline: 17.4235x; 1.1500x over previous
import dataclasses
import functools

import jax
import jax.numpy as jnp
from jax import lax
from jax.experimental import pallas as pl
from jax.experimental.pallas import tpu as pltpu
from jax.experimental.pallas import tpu_sc as plsc

N = 10000
E = 160000
D = 128
K = 32
C = 313
HEADS = 8
DEPTH = 8
NUM_CLASSES = 10

NPAD = 10240
EPAD = 180224
EW32 = EPAD // 32
EW16 = EPAD // 16
NROWS = NPAD // 128
DUMMY = N + 64
SPAD = 320
F32 = jnp.float32

_mesh = plsc.VectorSubcoreMesh(core_axis_name="c", subcore_axis_name="s")

_sc_params = pltpu.CompilerParams(use_tc_tiling_on_sc=False)
if "needs_layout_passes" in pltpu.CompilerParams.__dataclass_fields__:
    _sc_params = dataclasses.replace(_sc_params, needs_layout_passes=False)


def _edge_body(z_hbm, s_hbm, d_hbm, src2_hbm, dst2_hbm,
               iota_hbm, out_hbm,
               s_loc, d_loc, srcl_loc, dstl_loc, ebuf,
               den_loc, alpha_loc, gbuf0, gbuf1, sbuf0, sbuf1,
               mx_loc, mred_loc, iota_loc,
               out_sh, den_sh, mx_sh,
               gsem0, gsem1, ssem0, ssem1):
    cid = lax.axis_index("c")
    tid = lax.axis_index("s")

    scope = jax.named_scope
    pltpu.sync_copy(s_hbm, s_loc)
    pltpu.sync_copy(d_hbm, d_loc)
    pltpu.sync_copy(src2_hbm.at[2 * tid], srcl_loc.at[pl.ds(0, 44)])
    pltpu.sync_copy(src2_hbm.at[2 * tid + 1], srcl_loc.at[pl.ds(44, 44)])
    pltpu.sync_copy(dst2_hbm.at[2 * tid], dstl_loc.at[pl.ds(0, 44)])
    pltpu.sync_copy(dst2_hbm.at[2 * tid + 1], dstl_loc.at[pl.ds(44, 44)])
    pltpu.sync_copy(iota_hbm, iota_loc)

    zero16 = jnp.zeros((16,), F32)

    @pl.loop(0, NROWS)
    def _(r):
        for cc in range(8):
            den_loc[r, pl.ds(cc * 16, 16)] = zero16

    @pl.when(tid == 0)
    def _():
        pltpu.sync_copy(den_loc, den_sh)

    sc1 = scope("p1"); sc1.__enter__()
    mx_loc[...] = jnp.full((16,), -3e38, F32)

    @pl.loop(0, 88)
    def _(r):
        for cc in range(8):
            si = srcl_loc[r, pl.ds(cc * 16, 16)]
            di = dstl_loc[r, pl.ds(cc * 16, 16)]
            ev = plsc.load_gather(s_loc, [si]) + plsc.load_gather(d_loc, [di])
            ev = jnp.where(ev > 0, ev, 0.2 * ev)
            ebuf[r, pl.ds(cc * 16, 16)] = ev
            mx_loc[...] = jnp.maximum(mx_loc[...], ev)

    sc1.__exit__(None, None, None)
    sc2 = scope("p2"); sc2.__enter__()
    pltpu.sync_copy(mx_loc, mx_sh.at[tid])
    plsc.subcore_barrier()
    pltpu.sync_copy(mx_sh, mred_loc)
    mv = mred_loc[0, :]
    for t in range(1, 16):
        mv = jnp.maximum(mv, mred_loc[t, :])
    mscal = jnp.max(mv)
    mvec = jnp.full((16,), mscal, F32)

    @pl.loop(0, 88)
    def _(r):
        for cc in range(8):
            di = dstl_loc[r, pl.ds(cc * 16, 16)]
            ex = jnp.exp(ebuf[r, pl.ds(cc * 16, 16)] - mvec)
            plsc.addupdate_scatter(den_loc, [di >> 7, di & 127], ex)

    pltpu.sync_copy(den_loc, den_sh.at[iota_loc.at[0]], add=True)
    plsc.subcore_barrier()
    pltpu.sync_copy(den_sh, den_loc)

    @pl.loop(0, 44)
    def _(r):
        for cc in range(8):
            di = dstl_loc[cid * 44 + r, pl.ds(cc * 16, 16)]
            ex = jnp.exp(ebuf[cid * 44 + r, pl.ds(cc * 16, 16)] - mvec)
            dn = plsc.load_gather(den_loc, [di >> 7, di & 127])
            alpha_loc[pl.ds(r * 128 + cc * 16, 16)] = ex / (dn + 1e-16)

    sc2.__exit__(None, None, None)
    gbufs = (gbuf0, gbuf1)
    sbufs = (sbuf0, sbuf1)
    gsems = (gsem0, gsem1)
    ssems = (ssem0, ssem1)
    for h in range(4):
        scz = scope("hzero"); scz.__enter__()
        @pl.loop(0, 128)
        def _(r):
            for cc in range(2):
                gbuf0[r, pl.ds(cc * 16, 16)] = zero16
        for c5 in range(5):
            pltpu.sync_copy(gbuf0, out_sh.at[pl.ds(tid * 640 + c5 * 128, 128)])
        plsc.subcore_barrier()

        scz.__exit__(None, None, None)
        scc = scope("hchunks"); scc.__enter__()
        for b in range(2):
            pltpu.async_copy(z_hbm.at[h].at[srcl_loc.at[cid * 44 + b]],
                             gbufs[b], gsems[b])

        @pl.loop(0, 44, step=2)
        def _(jj):
            for b in range(2):
                j = jj + b
                pltpu.make_async_copy(
                    z_hbm.at[h].at[srcl_loc.at[cid * 44 + j]],
                    gbufs[b], gsems[b]).wait()

                @pl.when(jj > 0)
                def _():
                    pltpu.make_async_copy(
                        sbufs[b], out_sh.at[dstl_loc.at[cid * 44 + j]],
                        ssems[b]).wait()

                @pl.loop(0, 128, step=16)
                def _(r):
                    av16 = alpha_loc[pl.ds(j * 128 + r, 16)]
                    for k in range(16):
                        avk = lax.gather(
                            av16, jnp.full((16, 1), k, jnp.int32),
                            lax.GatherDimensionNumbers(
                                offset_dims=(), collapsed_slice_dims=(0,),
                                start_index_map=(0,)),
                            (1,), indices_are_sorted=True, unique_indices=False,
                            mode=lax.GatherScatterMode.PROMISE_IN_BOUNDS)
                        for cc in range(2):
                            sbufs[b][r + k, pl.ds(cc * 16, 16)] = (
                                gbufs[b][r + k, pl.ds(cc * 16, 16)] * avk)

                pltpu.async_copy(sbufs[b],
                                 out_sh.at[dstl_loc.at[cid * 44 + j]],
                                 ssems[b], add=True)

                @pl.when(j + 2 < 44)
                def _():
                    pltpu.async_copy(
                        z_hbm.at[h].at[srcl_loc.at[cid * 44 + j + 2]],
                        gbufs[b], gsems[b])

        for b in range(2):
            pltpu.make_async_copy(sbufs[b],
                                  out_sh.at[dstl_loc.at[cid * 44]],
                                  ssems[b]).wait()

        scc.__exit__(None, None, None)
        scw = scope("hwout"); scw.__enter__()
        plsc.subcore_barrier()
        for c5 in range(5):
            rows = pl.ds(tid * 640 + c5 * 128, 128)
            pltpu.sync_copy(out_sh.at[rows], out_hbm.at[cid].at[h].at[rows])
        scw.__exit__(None, None, None)


@jax.jit
def _edge_phase(z, s, d, src2, dst2, iota_rows):
    kfn = pl.kernel(
        _edge_body,
        out_type=jax.ShapeDtypeStruct((2, 4, NPAD, 32), F32),
        mesh=_mesh,
        compiler_params=_sc_params,
        scratch_types=[
            pltpu.VMEM((NPAD,), F32),
            pltpu.VMEM((NPAD,), F32),
            pltpu.VMEM((88, 128), jnp.int32),
            pltpu.VMEM((88, 128), jnp.int32),
            pltpu.VMEM((88, 128), F32),
            pltpu.VMEM((NROWS, 128), F32),
            pltpu.VMEM((EW32,), F32),
            pltpu.VMEM((128, 32), F32),
            pltpu.VMEM((128, 32), F32),
            pltpu.VMEM((128, 32), F32),
            pltpu.VMEM((128, 32), F32),
            pltpu.VMEM((16,), F32),
            pltpu.VMEM((16, 16), F32),
            pltpu.VMEM((1, NROWS), jnp.int32),
            pltpu.VMEM_SHARED((NPAD, 32), F32),
            pltpu.VMEM_SHARED((NROWS, 128), F32),
            pltpu.VMEM_SHARED((16, 16), F32),
            pltpu.SemaphoreType.DMA,
            pltpu.SemaphoreType.DMA,
            pltpu.SemaphoreType.DMA,
            pltpu.SemaphoreType.DMA,
        ],
    )
    return kfn(z, s, d, src2, dst2, iota_rows)


def _gather_body(h_hbm, idx_hbm, g_hbm, idx_loc, rowbuf):
    cid = lax.axis_index("c")
    tid = lax.axis_index("s")
    wid = tid * 2 + cid
    pltpu.sync_copy(idx_hbm.at[wid], idx_loc)
    for c in range(5):
        pltpu.sync_copy(h_hbm.at[idx_loc.at[c]], rowbuf)
        pltpu.sync_copy(rowbuf, g_hbm.at[pl.ds(wid * 320 + c * 64, 64)])


@jax.jit
def _cluster_gather(h2, idx):
    kfn = pl.kernel(
        _gather_body,
        out_type=jax.ShapeDtypeStruct((NPAD, D), F32),
        mesh=_mesh,
        scratch_types=[
            pltpu.VMEM((5, 64), jnp.int32),
            pltpu.VMEM((64, 128), F32),
        ],
    )
    return kfn(h2, idx)


_ROWS_BLK = 1024
_N_BLKS = NPAD // _ROWS_BLK


def _proj1_body(x_ref, wi_ref, bi_ref, w_ref, asrc_ref, adst_ref,
                z_ref, s_ref, d_ref):
    h = jnp.dot(x_ref[...], wi_ref[...], preferred_element_type=F32) + bi_ref[...]
    z = jnp.dot(h, w_ref[...], preferred_element_type=F32)
    for q in range(4):
        z_ref[q] = z[:, 32 * q:32 * q + 32]
    s_ref[...] = jnp.dot(z, asrc_ref[...], preferred_element_type=F32)
    d_ref[...] = jnp.dot(z, adst_ref[...], preferred_element_type=F32)


@jax.jit
def _proj1(x, w_in, b_in, w, a_src, a_dst):
    full = lambda i: (0, 0)
    return pl.pallas_call(
        _proj1_body,
        grid=(_N_BLKS,),
        in_specs=[
            pl.BlockSpec((_ROWS_BLK, D), lambda i: (i, 0)),
            pl.BlockSpec((D, D), full),
            pl.BlockSpec((1, D), full),
            pl.BlockSpec((D, D), full),
            pl.BlockSpec((D, 1), full),
            pl.BlockSpec((D, 1), full),
        ],
        out_specs=[
            pl.BlockSpec((4, _ROWS_BLK, 32), lambda i: (0, i, 0)),
            pl.BlockSpec((_ROWS_BLK, 1), lambda i: (i, 0)),
            pl.BlockSpec((_ROWS_BLK, 1), lambda i: (i, 0)),
        ],
        out_shape=[
            jax.ShapeDtypeStruct((4, NPAD, 32), F32),
            jax.ShapeDtypeStruct((NPAD, 1), F32),
            jax.ShapeDtypeStruct((NPAD, 1), F32),
        ],
    )(x, w_in, b_in[None, :], w, a_src[:, None], a_dst[:, None])


def _proj2_body(p_ref, b_ref, w_ref, asrc_ref, adst_ref, z_ref, s_ref, d_ref):
    h = jnp.concatenate([p_ref[0, q] + p_ref[1, q] for q in range(4)],
                        axis=-1) + b_ref[...]
    z = jnp.dot(h, w_ref[...], preferred_element_type=F32)
    for q in range(4):
        z_ref[q] = z[:, 32 * q:32 * q + 32]
    s_ref[...] = jnp.dot(z, asrc_ref[...], preferred_element_type=F32)
    d_ref[...] = jnp.dot(z, adst_ref[...], preferred_element_type=F32)


@jax.jit
def _proj2(p, b, w, a_src, a_dst):
    full = lambda i: (0, 0)
    return pl.pallas_call(
        _proj2_body,
        grid=(_N_BLKS,),
        in_specs=[
            pl.BlockSpec((2, 4, _ROWS_BLK, 32), lambda i: (0, 0, i, 0)),
            pl.BlockSpec((1, D), full),
            pl.BlockSpec((D, D), full),
            pl.BlockSpec((D, 1), full),
            pl.BlockSpec((D, 1), full),
        ],
        out_specs=[
            pl.BlockSpec((4, _ROWS_BLK, 32), lambda i: (0, i, 0)),
            pl.BlockSpec((_ROWS_BLK, 1), lambda i: (i, 0)),
            pl.BlockSpec((_ROWS_BLK, 1), lambda i: (i, 0)),
        ],
        out_shape=[
            jax.ShapeDtypeStruct((4, NPAD, 32), F32),
            jax.ShapeDtypeStruct((NPAD, 1), F32),
            jax.ShapeDtypeStruct((NPAD, 1), F32),
        ],
    )(p, b[None, :], w, a_src[:, None], a_dst[:, None])


def _sum2_body(p_ref, b_ref, h_ref):
    h_ref[...] = jnp.concatenate([p_ref[0, q] + p_ref[1, q]
                                  for q in range(4)], axis=-1) + b_ref[...]


@jax.jit
def _sum2(p, b):
    return pl.pallas_call(
        _sum2_body,
        grid=(_N_BLKS,),
        in_specs=[
            pl.BlockSpec((2, 4, _ROWS_BLK, 32), lambda i: (0, 0, i, 0)),
            pl.BlockSpec((1, D), lambda i: (0, 0)),
        ],
        out_specs=pl.BlockSpec((_ROWS_BLK, D), lambda i: (i, 0)),
        out_shape=jax.ShapeDtypeStruct((NPAD, D), F32),
    )(p, b[None, :])


def _emb_body(g_ref, m_ref, w_ref, b_ref, ce_ref, acc):
    k = pl.program_id(0)

    @pl.when(k == 0)
    def _():
        acc[...] = jnp.zeros_like(acc)

    acc[...] += jnp.dot(g_ref[0] * m_ref[0], w_ref[0],
                        preferred_element_type=F32)

    @pl.when(k == K - 1)
    def _():
        ce_ref[...] = acc[...] + b_ref[...]


@jax.jit
def _emb(g, m, w, b):
    return pl.pallas_call(
        _emb_body,
        grid=(K,),
        in_specs=[
            pl.BlockSpec((1, SPAD, D), lambda k: (k, 0, 0)),
            pl.BlockSpec((1, SPAD, 1), lambda k: (k, 0, 0)),
            pl.BlockSpec((1, D, D), lambda k: (k, 0, 0)),
            pl.BlockSpec((1, D), lambda k: (0, 0)),
        ],
        out_specs=pl.BlockSpec((SPAD, D), lambda k: (0, 0)),
        out_shape=jax.ShapeDtypeStruct((SPAD, D), F32),
        scratch_shapes=[pltpu.VMEM((SPAD, D), F32)],
    )(g, m, w, b[None, :])


def _ln(x, s, b):
    m = x.mean(-1, keepdims=True)
    v = ((x - m) ** 2).mean(-1, keepdims=True)
    return (x - m) / jnp.sqrt(v + 1e-5) * s + b


def _tf_body(y0_ref, l1s, l1b, wq, bq, wk, bk, wv, bv, wo, bo,
             l2s, l2b, wf1, bf1, wf2, bf2, wcls, bcls, out_ref, y):
    i = pl.program_id(0)

    @pl.when(i == 0)
    def _():
        y[...] = y0_ref[...]

    hd = D // HEADS
    colmask = lax.broadcasted_iota(jnp.int32, (SPAD, SPAD), 1) < (C + 1)

    t = _ln(y[...], l1s[0], l1b[0])
    q = jnp.dot(t, wq[0], preferred_element_type=F32) + bq[0]
    kk = jnp.dot(t, wk[0], preferred_element_type=F32) + bk[0]
    v = jnp.dot(t, wv[0], preferred_element_type=F32) + bv[0]
    ohs = []
    for h in range(HEADS):
        qh = q[:, h * hd:(h + 1) * hd]
        kh = kk[:, h * hd:(h + 1) * hd]
        vh = v[:, h * hd:(h + 1) * hd]
        sc = lax.dot_general(qh, kh, (((1,), (1,)), ((), ())),
                             preferred_element_type=F32) * (1.0 / 4.0)
        sc = jnp.where(colmask, sc, -1e30)
        sc = sc - sc.max(-1, keepdims=True)
        ex = jnp.exp(sc)
        att = ex / ex.sum(-1, keepdims=True)
        ohs.append(jnp.dot(att, vh, preferred_element_type=F32))
    o = jnp.concatenate(ohs, axis=1)
    ynew = y[...] + jnp.dot(o, wo[0], preferred_element_type=F32) + bo[0]
    t2 = _ln(ynew, l2s[0], l2b[0])
    f = jnp.dot(jax.nn.gelu(jnp.dot(t2, wf1[0], preferred_element_type=F32)
                            + bf1[0]),
                wf2[0], preferred_element_type=F32) + bf2[0]
    y[...] = ynew + f

    @pl.when(i == DEPTH - 1)
    def _():
        out_ref[...] = (jnp.dot(y[0:1, :], wcls[...],
                                preferred_element_type=F32) + bcls[...])


@jax.jit
def _transformer(y0, p):
    cube = lambda i: (i, 0, 0)
    fixed = lambda i: (0, 0)
    row3 = lambda n: pl.BlockSpec((1, 1, n), cube)
    r2 = lambda a: a[:, None, :]
    return pl.pallas_call(
        _tf_body,
        grid=(DEPTH,),
        in_specs=[
            pl.BlockSpec((SPAD, D), fixed),
            row3(D), row3(D),
            pl.BlockSpec((1, D, D), cube), row3(D),
            pl.BlockSpec((1, D, D), cube), row3(D),
            pl.BlockSpec((1, D, D), cube), row3(D),
            pl.BlockSpec((1, D, D), cube), row3(D),
            row3(D), row3(D),
            pl.BlockSpec((1, D, 4 * D), cube), row3(4 * D),
            pl.BlockSpec((1, 4 * D, D), cube), row3(D),
            pl.BlockSpec((D, NUM_CLASSES), fixed),
            pl.BlockSpec((1, NUM_CLASSES), fixed),
        ],
        out_specs=pl.BlockSpec((1, NUM_CLASSES), fixed),
        out_shape=jax.ShapeDtypeStruct((1, NUM_CLASSES), F32),
        scratch_shapes=[pltpu.VMEM((SPAD, D), F32)],
    )(y0, r2(p['ln1_s']), r2(p['ln1_b']), p['Wq'], r2(p['bq']),
      p['Wk'], r2(p['bk']), p['Wv'], r2(p['bv']), p['Wo'], r2(p['bo']),
      r2(p['ln2_s']), r2(p['ln2_b']),
      p['Wf1'], r2(p['bf1']), p['Wf2'], r2(p['bf2']),
      p['W_cls'], p['b_cls'][None, :])


def kernel(x, params, edge_index, cluster_indices, cluster_mask):
    p = params
    xp = jnp.pad(x, ((0, NPAD - N), (0, 0)))
    sl = jnp.arange(N, dtype=jnp.int32)
    npad_e = EPAD - E - N
    spread = N + (jnp.arange(npad_e, dtype=jnp.int32) % (NPAD - N))
    src = jnp.concatenate([edge_index[0].astype(jnp.int32), sl, spread])
    dst = jnp.concatenate([edge_index[1].astype(jnp.int32), sl, spread])
    src2 = src.reshape(32, 44, 128)
    dst2 = dst.reshape(32, 44, 128)
    iota_rows = jnp.arange(NROWS, dtype=jnp.int32).reshape(1, NROWS)

    z1, s1, d1 = _proj1(xp, p['W_in'], p['b_in'], p['W1'],
                        p['a_src1'], p['a_dst1'])
    pp1 = _edge_phase(z1, s1.reshape(-1), d1.reshape(-1),
                      src2, dst2, iota_rows)
    z2, s2, d2 = _proj2(pp1, p['b1'], p['W2'], p['a_src2'], p['a_dst2'])
    pp2 = _edge_phase(z2, s2.reshape(-1), d2.reshape(-1),
                      src2, dst2, iota_rows)
    h2 = _sum2(pp2, p['b2'])

    cit = jnp.pad(cluster_indices.T.astype(jnp.int32), ((0, 0), (0, 7)))
    idx = cit.reshape(32, 5, 64)
    mt = jnp.pad(cluster_mask.T, ((0, 0), (0, 7)))
    mbc = mt[:, :, None]

    g = _cluster_gather(h2, idx)
    ce = _emb(g.reshape(K, SPAD, D), mbc, p['W_emb'].reshape(K, D, D),
              p['b_emb'])
    y0 = jnp.concatenate([p['class_token'][None, :], ce[:C],
                          jnp.zeros((SPAD - C - 1, D), F32)], axis=0)
    return _transformer(y0, p)


# speedup vs baseline: 17.8271x; 1.0232x over previous
import dataclasses
import functools

import jax
import jax.numpy as jnp
from jax import lax
from jax.experimental import pallas as pl
from jax.experimental.pallas import tpu as pltpu
from jax.experimental.pallas import tpu_sc as plsc

N = 10000
E = 160000
D = 128
K = 32
C = 313
HEADS = 8
DEPTH = 8
NUM_CLASSES = 10

NPAD = 10240
EPAD = 180224
EW32 = EPAD // 32
EW16 = EPAD // 16
NROWS = NPAD // 128
DUMMY = N + 64
SPAD = 320
F32 = jnp.float32

_mesh = plsc.VectorSubcoreMesh(core_axis_name="c", subcore_axis_name="s")

_sc_params = pltpu.CompilerParams(use_tc_tiling_on_sc=False)
if "needs_layout_passes" in pltpu.CompilerParams.__dataclass_fields__:
    _sc_params = dataclasses.replace(_sc_params, needs_layout_passes=False)


def _edge_body(z_hbm, s_hbm, d_hbm, src2_hbm, dst2_hbm,
               iota_hbm, out_hbm,
               s_loc, d_loc, srcl_loc, dstl_loc, ebuf,
               den_loc, alpha_loc, gbuf0, gbuf1, sbuf0, sbuf1,
               mx_loc, mred_loc, iota_loc,
               out_sh, den_sh, mx_sh,
               gsem0, gsem1, ssem0, ssem1):
    cid = lax.axis_index("c")
    tid = lax.axis_index("s")

    scope = jax.named_scope
    pltpu.sync_copy(s_hbm, s_loc)
    pltpu.sync_copy(d_hbm, d_loc)
    pltpu.sync_copy(src2_hbm.at[2 * tid], srcl_loc.at[pl.ds(0, 44)])
    pltpu.sync_copy(src2_hbm.at[2 * tid + 1], srcl_loc.at[pl.ds(44, 44)])
    pltpu.sync_copy(dst2_hbm.at[2 * tid], dstl_loc.at[pl.ds(0, 44)])
    pltpu.sync_copy(dst2_hbm.at[2 * tid + 1], dstl_loc.at[pl.ds(44, 44)])
    pltpu.sync_copy(iota_hbm, iota_loc)

    zero16 = jnp.zeros((16,), F32)

    @pl.loop(0, NROWS)
    def _(r):
        for cc in range(8):
            den_loc[r, pl.ds(cc * 16, 16)] = zero16

    @pl.when(tid == 0)
    def _():
        pltpu.sync_copy(den_loc, den_sh)

    sc1 = scope("p1"); sc1.__enter__()
    mx_loc[...] = jnp.full((16,), -3e38, F32)

    @pl.loop(0, 88)
    def _(r):
        for cc in range(8):
            si = srcl_loc[r, pl.ds(cc * 16, 16)]
            di = dstl_loc[r, pl.ds(cc * 16, 16)]
            ev = (plsc.load_gather(s_loc, [si >> 7, si & 127])
                  + plsc.load_gather(d_loc, [di >> 7, di & 127]))
            ev = jnp.where(ev > 0, ev, 0.2 * ev)
            ebuf[r, pl.ds(cc * 16, 16)] = ev
            mx_loc[...] = jnp.maximum(mx_loc[...], ev)

    sc1.__exit__(None, None, None)
    sc2 = scope("p2"); sc2.__enter__()
    pltpu.sync_copy(mx_loc, mx_sh.at[tid])
    plsc.subcore_barrier()
    pltpu.sync_copy(mx_sh, mred_loc)
    mv = mred_loc[0, :]
    for t in range(1, 16):
        mv = jnp.maximum(mv, mred_loc[t, :])
    mscal = jnp.max(mv)
    mvec = jnp.full((16,), mscal, F32)

    @pl.loop(0, 88)
    def _(r):
        for cc in range(8):
            di = dstl_loc[r, pl.ds(cc * 16, 16)]
            ex = jnp.exp(ebuf[r, pl.ds(cc * 16, 16)] - mvec)
            plsc.addupdate_scatter(den_loc, [di >> 7, di & 127], ex)

    pltpu.sync_copy(den_loc, den_sh.at[iota_loc.at[0]], add=True)
    plsc.subcore_barrier()
    pltpu.sync_copy(den_sh, den_loc)

    @pl.loop(0, 44)
    def _(r):
        for cc in range(8):
            di = dstl_loc[cid * 44 + r, pl.ds(cc * 16, 16)]
            ex = jnp.exp(ebuf[cid * 44 + r, pl.ds(cc * 16, 16)] - mvec)
            dn = plsc.load_gather(den_loc, [di >> 7, di & 127])
            alpha_loc[pl.ds(r * 128 + cc * 16, 16)] = ex / (dn + 1e-16)

    sc2.__exit__(None, None, None)
    gbufs = (gbuf0, gbuf1)
    sbufs = (sbuf0, sbuf1)
    gsems = (gsem0, gsem1)
    ssems = (ssem0, ssem1)
    for h in range(4):
        scz = scope("hzero"); scz.__enter__()
        @pl.loop(0, 128)
        def _(r):
            for cc in range(2):
                gbuf0[r, pl.ds(cc * 16, 16)] = zero16
        for c5 in range(5):
            pltpu.sync_copy(gbuf0, out_sh.at[pl.ds(tid * 640 + c5 * 128, 128)])
        plsc.subcore_barrier()

        scz.__exit__(None, None, None)
        scc = scope("hchunks"); scc.__enter__()
        for b in range(2):
            pltpu.async_copy(z_hbm.at[h].at[srcl_loc.at[cid * 44 + b]],
                             gbufs[b], gsems[b])

        @pl.loop(0, 44, step=2)
        def _(jj):
            for b in range(2):
                j = jj + b
                pltpu.make_async_copy(
                    z_hbm.at[h].at[srcl_loc.at[cid * 44 + j]],
                    gbufs[b], gsems[b]).wait()

                @pl.when(jj > 0)
                def _():
                    pltpu.make_async_copy(
                        sbufs[b], out_sh.at[dstl_loc.at[cid * 44 + j]],
                        ssems[b]).wait()

                @pl.loop(0, 128, step=16)
                def _(r):
                    av16 = alpha_loc[pl.ds(j * 128 + r, 16)]
                    for k in range(16):
                        avk = lax.gather(
                            av16, jnp.full((16, 1), k, jnp.int32),
                            lax.GatherDimensionNumbers(
                                offset_dims=(), collapsed_slice_dims=(0,),
                                start_index_map=(0,)),
                            (1,), indices_are_sorted=True, unique_indices=False,
                            mode=lax.GatherScatterMode.PROMISE_IN_BOUNDS)
                        for cc in range(2):
                            sbufs[b][r + k, pl.ds(cc * 16, 16)] = (
                                gbufs[b][r + k, pl.ds(cc * 16, 16)] * avk)

                pltpu.async_copy(sbufs[b],
                                 out_sh.at[dstl_loc.at[cid * 44 + j]],
                                 ssems[b], add=True)

                @pl.when(j + 2 < 44)
                def _():
                    pltpu.async_copy(
                        z_hbm.at[h].at[srcl_loc.at[cid * 44 + j + 2]],
                        gbufs[b], gsems[b])

        for b in range(2):
            pltpu.make_async_copy(sbufs[b],
                                  out_sh.at[dstl_loc.at[cid * 44]],
                                  ssems[b]).wait()

        scc.__exit__(None, None, None)
        scw = scope("hwout"); scw.__enter__()
        plsc.subcore_barrier()
        for c5 in range(5):
            rows = pl.ds(tid * 640 + c5 * 128, 128)
            pltpu.sync_copy(out_sh.at[rows], out_hbm.at[cid].at[h].at[rows])
        scw.__exit__(None, None, None)


@jax.jit
def _edge_phase(z, s, d, src2, dst2, iota_rows):
    kfn = pl.kernel(
        _edge_body,
        out_type=jax.ShapeDtypeStruct((2, 4, NPAD, 32), F32),
        mesh=_mesh,
        compiler_params=_sc_params,
        scratch_types=[
            pltpu.VMEM((NROWS, 128), F32),
            pltpu.VMEM((NROWS, 128), F32),
            pltpu.VMEM((88, 128), jnp.int32),
            pltpu.VMEM((88, 128), jnp.int32),
            pltpu.VMEM((88, 128), F32),
            pltpu.VMEM((NROWS, 128), F32),
            pltpu.VMEM((EW32,), F32),
            pltpu.VMEM((128, 32), F32),
            pltpu.VMEM((128, 32), F32),
            pltpu.VMEM((128, 32), F32),
            pltpu.VMEM((128, 32), F32),
            pltpu.VMEM((16,), F32),
            pltpu.VMEM((16, 16), F32),
            pltpu.VMEM((1, NROWS), jnp.int32),
            pltpu.VMEM_SHARED((NPAD, 32), F32),
            pltpu.VMEM_SHARED((NROWS, 128), F32),
            pltpu.VMEM_SHARED((16, 16), F32),
            pltpu.SemaphoreType.DMA,
            pltpu.SemaphoreType.DMA,
            pltpu.SemaphoreType.DMA,
            pltpu.SemaphoreType.DMA,
        ],
    )
    return kfn(z, s, d, src2, dst2, iota_rows)


def _gather_body(h_hbm, idx_hbm, g_hbm, idx_loc, rowbuf):
    cid = lax.axis_index("c")
    tid = lax.axis_index("s")
    wid = tid * 2 + cid
    pltpu.sync_copy(idx_hbm.at[wid], idx_loc)
    for c in range(5):
        pltpu.sync_copy(h_hbm.at[idx_loc.at[c]], rowbuf)
        pltpu.sync_copy(rowbuf, g_hbm.at[pl.ds(wid * 320 + c * 64, 64)])


@jax.jit
def _cluster_gather(h2, idx):
    kfn = pl.kernel(
        _gather_body,
        out_type=jax.ShapeDtypeStruct((NPAD, D), F32),
        mesh=_mesh,
        scratch_types=[
            pltpu.VMEM((5, 64), jnp.int32),
            pltpu.VMEM((64, 128), F32),
        ],
    )
    return kfn(h2, idx)


_ROWS_BLK = 1024
_N_BLKS = NPAD // _ROWS_BLK


def _proj1_body(x_ref, wi_ref, bi_ref, w_ref, asrc_ref, adst_ref,
                z_ref, s_ref, d_ref):
    h = jnp.dot(x_ref[...], wi_ref[...], preferred_element_type=F32) + bi_ref[...]
    z = jnp.dot(h, w_ref[...], preferred_element_type=F32)
    for q in range(4):
        z_ref[q] = z[:, 32 * q:32 * q + 32]
    s_ref[...] = jnp.dot(z, asrc_ref[...],
                         preferred_element_type=F32).reshape(8, 128)
    d_ref[...] = jnp.dot(z, adst_ref[...],
                         preferred_element_type=F32).reshape(8, 128)


@jax.jit
def _proj1(x, w_in, b_in, w, a_src, a_dst):
    full = lambda i: (0, 0)
    return pl.pallas_call(
        _proj1_body,
        grid=(_N_BLKS,),
        in_specs=[
            pl.BlockSpec((_ROWS_BLK, D), lambda i: (i, 0)),
            pl.BlockSpec((D, D), full),
            pl.BlockSpec((1, D), full),
            pl.BlockSpec((D, D), full),
            pl.BlockSpec((D, 1), full),
            pl.BlockSpec((D, 1), full),
        ],
        out_specs=[
            pl.BlockSpec((4, _ROWS_BLK, 32), lambda i: (0, i, 0)),
            pl.BlockSpec((8, 128), lambda i: (i, 0)),
            pl.BlockSpec((8, 128), lambda i: (i, 0)),
        ],
        out_shape=[
            jax.ShapeDtypeStruct((4, NPAD, 32), F32),
            jax.ShapeDtypeStruct((NROWS, 128), F32),
            jax.ShapeDtypeStruct((NROWS, 128), F32),
        ],
    )(x, w_in, b_in[None, :], w, a_src[:, None], a_dst[:, None])


def _proj2_body(p_ref, b_ref, w_ref, asrc_ref, adst_ref, z_ref, s_ref, d_ref):
    h = jnp.concatenate([p_ref[0, q] + p_ref[1, q] for q in range(4)],
                        axis=-1) + b_ref[...]
    z = jnp.dot(h, w_ref[...], preferred_element_type=F32)
    for q in range(4):
        z_ref[q] = z[:, 32 * q:32 * q + 32]
    s_ref[...] = jnp.dot(z, asrc_ref[...],
                         preferred_element_type=F32).reshape(8, 128)
    d_ref[...] = jnp.dot(z, adst_ref[...],
                         preferred_element_type=F32).reshape(8, 128)


@jax.jit
def _proj2(p, b, w, a_src, a_dst):
    full = lambda i: (0, 0)
    return pl.pallas_call(
        _proj2_body,
        grid=(_N_BLKS,),
        in_specs=[
            pl.BlockSpec((2, 4, _ROWS_BLK, 32), lambda i: (0, 0, i, 0)),
            pl.BlockSpec((1, D), full),
            pl.BlockSpec((D, D), full),
            pl.BlockSpec((D, 1), full),
            pl.BlockSpec((D, 1), full),
        ],
        out_specs=[
            pl.BlockSpec((4, _ROWS_BLK, 32), lambda i: (0, i, 0)),
            pl.BlockSpec((8, 128), lambda i: (i, 0)),
            pl.BlockSpec((8, 128), lambda i: (i, 0)),
        ],
        out_shape=[
            jax.ShapeDtypeStruct((4, NPAD, 32), F32),
            jax.ShapeDtypeStruct((NROWS, 128), F32),
            jax.ShapeDtypeStruct((NROWS, 128), F32),
        ],
    )(p, b[None, :], w, a_src[:, None], a_dst[:, None])


def _sum2_body(p_ref, b_ref, h_ref):
    h_ref[...] = jnp.concatenate([p_ref[0, q] + p_ref[1, q]
                                  for q in range(4)], axis=-1) + b_ref[...]


@jax.jit
def _sum2(p, b):
    return pl.pallas_call(
        _sum2_body,
        grid=(_N_BLKS,),
        in_specs=[
            pl.BlockSpec((2, 4, _ROWS_BLK, 32), lambda i: (0, 0, i, 0)),
            pl.BlockSpec((1, D), lambda i: (0, 0)),
        ],
        out_specs=pl.BlockSpec((_ROWS_BLK, D), lambda i: (i, 0)),
        out_shape=jax.ShapeDtypeStruct((NPAD, D), F32),
    )(p, b[None, :])


def _emb_body(g_ref, m_ref, w_ref, b_ref, ce_ref, acc):
    k = pl.program_id(0)

    @pl.when(k == 0)
    def _():
        acc[...] = jnp.zeros_like(acc)

    acc[...] += jnp.dot(g_ref[0] * m_ref[0], w_ref[0],
                        preferred_element_type=F32)

    @pl.when(k == K - 1)
    def _():
        ce_ref[...] = acc[...] + b_ref[...]


@jax.jit
def _emb(g, m, w, b):
    return pl.pallas_call(
        _emb_body,
        grid=(K,),
        in_specs=[
            pl.BlockSpec((1, SPAD, D), lambda k: (k, 0, 0)),
            pl.BlockSpec((1, SPAD, 1), lambda k: (k, 0, 0)),
            pl.BlockSpec((1, D, D), lambda k: (k, 0, 0)),
            pl.BlockSpec((1, D), lambda k: (0, 0)),
        ],
        out_specs=pl.BlockSpec((SPAD, D), lambda k: (0, 0)),
        out_shape=jax.ShapeDtypeStruct((SPAD, D), F32),
        scratch_shapes=[pltpu.VMEM((SPAD, D), F32)],
    )(g, m, w, b[None, :])


def _ln(x, s, b):
    m = x.mean(-1, keepdims=True)
    v = ((x - m) ** 2).mean(-1, keepdims=True)
    return (x - m) / jnp.sqrt(v + 1e-5) * s + b


def _tf_body(y0_ref, l1s, l1b, wq, bq, wk, bk, wv, bv, wo, bo,
             l2s, l2b, wf1, bf1, wf2, bf2, wcls, bcls, out_ref, y):
    i = pl.program_id(0)

    @pl.when(i == 0)
    def _():
        y[...] = y0_ref[...]

    hd = D // HEADS
    colmask = lax.broadcasted_iota(jnp.int32, (SPAD, SPAD), 1) < (C + 1)

    t = _ln(y[...], l1s[0], l1b[0])
    q = jnp.dot(t, wq[0], preferred_element_type=F32) + bq[0]
    kk = jnp.dot(t, wk[0], preferred_element_type=F32) + bk[0]
    v = jnp.dot(t, wv[0], preferred_element_type=F32) + bv[0]
    ohs = []
    for h in range(HEADS):
        qh = q[:, h * hd:(h + 1) * hd]
        kh = kk[:, h * hd:(h + 1) * hd]
        vh = v[:, h * hd:(h + 1) * hd]
        sc = lax.dot_general(qh, kh, (((1,), (1,)), ((), ())),
                             preferred_element_type=F32) * (1.0 / 4.0)
        sc = jnp.where(colmask, sc, -1e30)
        sc = sc - sc.max(-1, keepdims=True)
        ex = jnp.exp(sc)
        att = ex / ex.sum(-1, keepdims=True)
        ohs.append(jnp.dot(att, vh, preferred_element_type=F32))
    o = jnp.concatenate(ohs, axis=1)
    ynew = y[...] + jnp.dot(o, wo[0], preferred_element_type=F32) + bo[0]
    t2 = _ln(ynew, l2s[0], l2b[0])
    f = jnp.dot(jax.nn.gelu(jnp.dot(t2, wf1[0], preferred_element_type=F32)
                            + bf1[0]),
                wf2[0], preferred_element_type=F32) + bf2[0]
    y[...] = ynew + f

    @pl.when(i == DEPTH - 1)
    def _():
        out_ref[...] = (jnp.dot(y[0:1, :], wcls[...],
                                preferred_element_type=F32) + bcls[...])


@jax.jit
def _transformer(y0, p):
    cube = lambda i: (i, 0, 0)
    fixed = lambda i: (0, 0)
    row3 = lambda n: pl.BlockSpec((1, 1, n), cube)
    r2 = lambda a: a[:, None, :]
    return pl.pallas_call(
        _tf_body,
        grid=(DEPTH,),
        in_specs=[
            pl.BlockSpec((SPAD, D), fixed),
            row3(D), row3(D),
            pl.BlockSpec((1, D, D), cube), row3(D),
            pl.BlockSpec((1, D, D), cube), row3(D),
            pl.BlockSpec((1, D, D), cube), row3(D),
            pl.BlockSpec((1, D, D), cube), row3(D),
            row3(D), row3(D),
            pl.BlockSpec((1, D, 4 * D), cube), row3(4 * D),
            pl.BlockSpec((1, 4 * D, D), cube), row3(D),
            pl.BlockSpec((D, NUM_CLASSES), fixed),
            pl.BlockSpec((1, NUM_CLASSES), fixed),
        ],
        out_specs=pl.BlockSpec((1, NUM_CLASSES), fixed),
        out_shape=jax.ShapeDtypeStruct((1, NUM_CLASSES), F32),
        scratch_shapes=[pltpu.VMEM((SPAD, D), F32)],
    )(y0, r2(p['ln1_s']), r2(p['ln1_b']), p['Wq'], r2(p['bq']),
      p['Wk'], r2(p['bk']), p['Wv'], r2(p['bv']), p['Wo'], r2(p['bo']),
      r2(p['ln2_s']), r2(p['ln2_b']),
      p['Wf1'], r2(p['bf1']), p['Wf2'], r2(p['bf2']),
      p['W_cls'], p['b_cls'][None, :])


def kernel(x, params, edge_index, cluster_indices, cluster_mask):
    p = params
    xp = jnp.pad(x, ((0, NPAD - N), (0, 0)))
    sl = jnp.arange(N, dtype=jnp.int32)
    npad_e = EPAD - E - N
    spread = N + (jnp.arange(npad_e, dtype=jnp.int32) % (NPAD - N))
    src = jnp.concatenate([edge_index[0].astype(jnp.int32), sl, spread])
    dst = jnp.concatenate([edge_index[1].astype(jnp.int32), sl, spread])
    src2 = src.reshape(32, 44, 128)
    dst2 = dst.reshape(32, 44, 128)
    iota_rows = jnp.arange(NROWS, dtype=jnp.int32).reshape(1, NROWS)

    z1, s1, d1 = _proj1(xp, p['W_in'], p['b_in'], p['W1'],
                        p['a_src1'], p['a_dst1'])
    pp1 = _edge_phase(z1, s1, d1, src2, dst2, iota_rows)
    z2, s2, d2 = _proj2(pp1, p['b1'], p['W2'], p['a_src2'], p['a_dst2'])
    pp2 = _edge_phase(z2, s2, d2, src2, dst2, iota_rows)
    h2 = _sum2(pp2, p['b2'])

    cit = jnp.pad(cluster_indices.T.astype(jnp.int32), ((0, 0), (0, 7)))
    idx = cit.reshape(32, 5, 64)
    mt = jnp.pad(cluster_mask.T, ((0, 0), (0, 7)))
    mbc = mt[:, :, None]

    g = _cluster_gather(h2, idx)
    ce = _emb(g.reshape(K, SPAD, D), mbc, p['W_emb'].reshape(K, D, D),
              p['b_emb'])
    y0 = jnp.concatenate([p['class_token'][None, :], ce[:C],
                          jnp.zeros((SPAD - C - 1, D), F32)], axis=0)
    return _transformer(y0, p)


# speedup vs baseline: 17.8456x; 1.0010x over previous
import dataclasses
import functools

import jax
import jax.numpy as jnp
from jax import lax
from jax.experimental import pallas as pl
from jax.experimental.pallas import tpu as pltpu
from jax.experimental.pallas import tpu_sc as plsc

N = 10000
E = 160000
D = 128
K = 32
C = 313
HEADS = 8
DEPTH = 8
NUM_CLASSES = 10

NPAD = 10240
EPAD = 180224
EW32 = EPAD // 32
EW16 = EPAD // 16
NROWS = NPAD // 128
DUMMY = N + 64
SPAD = 320
F32 = jnp.float32

_mesh = plsc.VectorSubcoreMesh(core_axis_name="c", subcore_axis_name="s")

_sc_params = pltpu.CompilerParams(use_tc_tiling_on_sc=False)
if "needs_layout_passes" in pltpu.CompilerParams.__dataclass_fields__:
    _sc_params = dataclasses.replace(_sc_params, needs_layout_passes=False)


def _edge_body(z_hbm, s_hbm, d_hbm, src2_hbm, dst2_hbm,
               iota_hbm, out_hbm,
               s_loc, d_loc, srcl_loc, dstl_loc, ebuf,
               den_loc, alpha_loc, gbuf0, gbuf1, sbuf0, sbuf1,
               mx_loc, mred_loc, iota_loc,
               out_sh, den_sh, mx_sh,
               gsem0, gsem1, ssem0, ssem1):
    cid = lax.axis_index("c")
    tid = lax.axis_index("s")

    pltpu.sync_copy(s_hbm, s_loc)
    pltpu.sync_copy(d_hbm, d_loc)
    pltpu.sync_copy(src2_hbm.at[2 * tid], srcl_loc.at[pl.ds(0, 44)])
    pltpu.sync_copy(src2_hbm.at[2 * tid + 1], srcl_loc.at[pl.ds(44, 44)])
    pltpu.sync_copy(dst2_hbm.at[2 * tid], dstl_loc.at[pl.ds(0, 44)])
    pltpu.sync_copy(dst2_hbm.at[2 * tid + 1], dstl_loc.at[pl.ds(44, 44)])
    pltpu.sync_copy(iota_hbm, iota_loc)

    zero16 = jnp.zeros((16,), F32)

    @pl.loop(0, NROWS)
    def _(r):
        for cc in range(8):
            den_loc[r, pl.ds(cc * 16, 16)] = zero16

    @pl.when(tid == 0)
    def _():
        pltpu.sync_copy(den_loc, den_sh)

    mx_loc[...] = jnp.full((16,), -3e38, F32)

    @pl.loop(0, 88)
    def _(r):
        for cc in range(8):
            si = srcl_loc[r, pl.ds(cc * 16, 16)]
            di = dstl_loc[r, pl.ds(cc * 16, 16)]
            ev = (plsc.load_gather(s_loc, [si >> 7, si & 127])
                  + plsc.load_gather(d_loc, [di >> 7, di & 127]))
            ev = jnp.where(ev > 0, ev, 0.2 * ev)
            ebuf[r, pl.ds(cc * 16, 16)] = ev
            mx_loc[...] = jnp.maximum(mx_loc[...], ev)

    pltpu.sync_copy(mx_loc, mx_sh.at[tid])
    plsc.subcore_barrier()
    pltpu.sync_copy(mx_sh, mred_loc)
    mv = mred_loc[0, :]
    for t in range(1, 16):
        mv = jnp.maximum(mv, mred_loc[t, :])
    mscal = jnp.max(mv)
    mvec = jnp.full((16,), mscal, F32)

    @pl.loop(0, 88)
    def _(r):
        for cc in range(8):
            di = dstl_loc[r, pl.ds(cc * 16, 16)]
            ex = jnp.exp(ebuf[r, pl.ds(cc * 16, 16)] - mvec)
            plsc.addupdate_scatter(den_loc, [di >> 7, di & 127], ex)

    pltpu.sync_copy(den_loc, den_sh.at[iota_loc.at[0]], add=True)
    plsc.subcore_barrier()
    pltpu.sync_copy(den_sh, den_loc)

    @pl.loop(0, 44)
    def _(r):
        for cc in range(8):
            di = dstl_loc[cid * 44 + r, pl.ds(cc * 16, 16)]
            ex = jnp.exp(ebuf[cid * 44 + r, pl.ds(cc * 16, 16)] - mvec)
            dn = plsc.load_gather(den_loc, [di >> 7, di & 127])
            alpha_loc[pl.ds(r * 128 + cc * 16, 16)] = ex / (dn + 1e-16)

    gbufs = (gbuf0, gbuf1)
    sbufs = (sbuf0, sbuf1)
    gsems = (gsem0, gsem1)
    ssems = (ssem0, ssem1)
    for h in range(4):
        @pl.loop(0, 128)
        def _(r):
            for cc in range(2):
                gbuf0[r, pl.ds(cc * 16, 16)] = zero16
        for c5 in range(5):
            pltpu.sync_copy(gbuf0, out_sh.at[pl.ds(tid * 640 + c5 * 128, 128)])
        plsc.subcore_barrier()

        for b in range(2):
            pltpu.async_copy(z_hbm.at[h].at[srcl_loc.at[cid * 44 + b]],
                             gbufs[b], gsems[b])

        @pl.loop(0, 44, step=2)
        def _(jj):
            for b in range(2):
                j = jj + b
                pltpu.make_async_copy(
                    z_hbm.at[h].at[srcl_loc.at[cid * 44 + j]],
                    gbufs[b], gsems[b]).wait()

                @pl.when(jj > 0)
                def _():
                    pltpu.make_async_copy(
                        sbufs[b], out_sh.at[dstl_loc.at[cid * 44 + j]],
                        ssems[b]).wait()

                @pl.loop(0, 128, step=16)
                def _(r):
                    av16 = alpha_loc[pl.ds(j * 128 + r, 16)]
                    for k in range(16):
                        avk = lax.gather(
                            av16, jnp.full((16, 1), k, jnp.int32),
                            lax.GatherDimensionNumbers(
                                offset_dims=(), collapsed_slice_dims=(0,),
                                start_index_map=(0,)),
                            (1,), indices_are_sorted=True, unique_indices=False,
                            mode=lax.GatherScatterMode.PROMISE_IN_BOUNDS)
                        for cc in range(2):
                            sbufs[b][r + k, pl.ds(cc * 16, 16)] = (
                                gbufs[b][r + k, pl.ds(cc * 16, 16)] * avk)

                pltpu.async_copy(sbufs[b],
                                 out_sh.at[dstl_loc.at[cid * 44 + j]],
                                 ssems[b], add=True)

                @pl.when(j + 2 < 44)
                def _():
                    pltpu.async_copy(
                        z_hbm.at[h].at[srcl_loc.at[cid * 44 + j + 2]],
                        gbufs[b], gsems[b])

        for b in range(2):
            pltpu.make_async_copy(sbufs[b],
                                  out_sh.at[dstl_loc.at[cid * 44]],
                                  ssems[b]).wait()

        plsc.subcore_barrier()
        for c5 in range(5):
            rows = pl.ds(tid * 640 + c5 * 128, 128)
            pltpu.sync_copy(out_sh.at[rows], out_hbm.at[cid].at[h].at[rows])


@jax.jit
def _edge_phase(z, s, d, src2, dst2, iota_rows):
    kfn = pl.kernel(
        _edge_body,
        out_type=jax.ShapeDtypeStruct((2, 4, NPAD, 32), F32),
        mesh=_mesh,
        compiler_params=_sc_params,
        scratch_types=[
            pltpu.VMEM((NROWS, 128), F32),
            pltpu.VMEM((NROWS, 128), F32),
            pltpu.VMEM((88, 128), jnp.int32),
            pltpu.VMEM((88, 128), jnp.int32),
            pltpu.VMEM((88, 128), F32),
            pltpu.VMEM((NROWS, 128), F32),
            pltpu.VMEM((EW32,), F32),
            pltpu.VMEM((128, 32), F32),
            pltpu.VMEM((128, 32), F32),
            pltpu.VMEM((128, 32), F32),
            pltpu.VMEM((128, 32), F32),
            pltpu.VMEM((16,), F32),
            pltpu.VMEM((16, 16), F32),
            pltpu.VMEM((1, NROWS), jnp.int32),
            pltpu.VMEM_SHARED((NPAD, 32), F32),
            pltpu.VMEM_SHARED((NROWS, 128), F32),
            pltpu.VMEM_SHARED((16, 16), F32),
            pltpu.SemaphoreType.DMA,
            pltpu.SemaphoreType.DMA,
            pltpu.SemaphoreType.DMA,
            pltpu.SemaphoreType.DMA,
        ],
    )
    return kfn(z, s, d, src2, dst2, iota_rows)


def _gather_body(h_hbm, idx_hbm, g_hbm, idx_loc, rowbuf):
    cid = lax.axis_index("c")
    tid = lax.axis_index("s")
    wid = tid * 2 + cid
    pltpu.sync_copy(idx_hbm.at[wid], idx_loc)
    for c in range(5):
        pltpu.sync_copy(h_hbm.at[idx_loc.at[c]], rowbuf)
        pltpu.sync_copy(rowbuf, g_hbm.at[pl.ds(wid * 320 + c * 64, 64)])


@jax.jit
def _cluster_gather(h2, idx):
    kfn = pl.kernel(
        _gather_body,
        out_type=jax.ShapeDtypeStruct((NPAD, D), F32),
        mesh=_mesh,
        scratch_types=[
            pltpu.VMEM((5, 64), jnp.int32),
            pltpu.VMEM((64, 128), F32),
        ],
    )
    return kfn(h2, idx)


_ROWS_BLK = 1024
_N_BLKS = NPAD // _ROWS_BLK


def _proj1_body(x_ref, wi_ref, bi_ref, w_ref, asrc_ref, adst_ref,
                z_ref, s_ref, d_ref):
    h = jnp.dot(x_ref[...], wi_ref[...], preferred_element_type=F32) + bi_ref[...]
    z = jnp.dot(h, w_ref[...], preferred_element_type=F32)
    for q in range(4):
        z_ref[q] = z[:, 32 * q:32 * q + 32]
    s_ref[...] = jnp.dot(z, asrc_ref[...],
                         preferred_element_type=F32).reshape(8, 128)
    d_ref[...] = jnp.dot(z, adst_ref[...],
                         preferred_element_type=F32).reshape(8, 128)


@jax.jit
def _proj1(x, w_in, b_in, w, a_src, a_dst):
    full = lambda i: (0, 0)
    return pl.pallas_call(
        _proj1_body,
        grid=(_N_BLKS,),
        in_specs=[
            pl.BlockSpec((_ROWS_BLK, D), lambda i: (i, 0)),
            pl.BlockSpec((D, D), full),
            pl.BlockSpec((1, D), full),
            pl.BlockSpec((D, D), full),
            pl.BlockSpec((D, 1), full),
            pl.BlockSpec((D, 1), full),
        ],
        out_specs=[
            pl.BlockSpec((4, _ROWS_BLK, 32), lambda i: (0, i, 0)),
            pl.BlockSpec((8, 128), lambda i: (i, 0)),
            pl.BlockSpec((8, 128), lambda i: (i, 0)),
        ],
        out_shape=[
            jax.ShapeDtypeStruct((4, NPAD, 32), F32),
            jax.ShapeDtypeStruct((NROWS, 128), F32),
            jax.ShapeDtypeStruct((NROWS, 128), F32),
        ],
    )(x, w_in, b_in[None, :], w, a_src[:, None], a_dst[:, None])


def _proj2_body(p_ref, b_ref, w_ref, asrc_ref, adst_ref, z_ref, s_ref, d_ref):
    h = jnp.concatenate([p_ref[0, q] + p_ref[1, q] for q in range(4)],
                        axis=-1) + b_ref[...]
    z = jnp.dot(h, w_ref[...], preferred_element_type=F32)
    for q in range(4):
        z_ref[q] = z[:, 32 * q:32 * q + 32]
    s_ref[...] = jnp.dot(z, asrc_ref[...],
                         preferred_element_type=F32).reshape(8, 128)
    d_ref[...] = jnp.dot(z, adst_ref[...],
                         preferred_element_type=F32).reshape(8, 128)


@jax.jit
def _proj2(p, b, w, a_src, a_dst):
    full = lambda i: (0, 0)
    return pl.pallas_call(
        _proj2_body,
        grid=(_N_BLKS,),
        in_specs=[
            pl.BlockSpec((2, 4, _ROWS_BLK, 32), lambda i: (0, 0, i, 0)),
            pl.BlockSpec((1, D), full),
            pl.BlockSpec((D, D), full),
            pl.BlockSpec((D, 1), full),
            pl.BlockSpec((D, 1), full),
        ],
        out_specs=[
            pl.BlockSpec((4, _ROWS_BLK, 32), lambda i: (0, i, 0)),
            pl.BlockSpec((8, 128), lambda i: (i, 0)),
            pl.BlockSpec((8, 128), lambda i: (i, 0)),
        ],
        out_shape=[
            jax.ShapeDtypeStruct((4, NPAD, 32), F32),
            jax.ShapeDtypeStruct((NROWS, 128), F32),
            jax.ShapeDtypeStruct((NROWS, 128), F32),
        ],
    )(p, b[None, :], w, a_src[:, None], a_dst[:, None])


def _sum2_body(p_ref, b_ref, h_ref):
    h_ref[...] = jnp.concatenate([p_ref[0, q] + p_ref[1, q]
                                  for q in range(4)], axis=-1) + b_ref[...]


@jax.jit
def _sum2(p, b):
    return pl.pallas_call(
        _sum2_body,
        grid=(_N_BLKS,),
        in_specs=[
            pl.BlockSpec((2, 4, _ROWS_BLK, 32), lambda i: (0, 0, i, 0)),
            pl.BlockSpec((1, D), lambda i: (0, 0)),
        ],
        out_specs=pl.BlockSpec((_ROWS_BLK, D), lambda i: (i, 0)),
        out_shape=jax.ShapeDtypeStruct((NPAD, D), F32),
    )(p, b[None, :])


def _emb_body(g_ref, m_ref, w_ref, b_ref, ce_ref, acc):
    k = pl.program_id(0)

    @pl.when(k == 0)
    def _():
        acc[...] = jnp.zeros_like(acc)

    acc[...] += jnp.dot(g_ref[0] * m_ref[0], w_ref[0],
                        preferred_element_type=F32)

    @pl.when(k == K - 1)
    def _():
        ce_ref[...] = acc[...] + b_ref[...]


@jax.jit
def _emb(g, m, w, b):
    return pl.pallas_call(
        _emb_body,
        grid=(K,),
        in_specs=[
            pl.BlockSpec((1, SPAD, D), lambda k: (k, 0, 0)),
            pl.BlockSpec((1, SPAD, 1), lambda k: (k, 0, 0)),
            pl.BlockSpec((1, D, D), lambda k: (k, 0, 0)),
            pl.BlockSpec((1, D), lambda k: (0, 0)),
        ],
        out_specs=pl.BlockSpec((SPAD, D), lambda k: (0, 0)),
        out_shape=jax.ShapeDtypeStruct((SPAD, D), F32),
        scratch_shapes=[pltpu.VMEM((SPAD, D), F32)],
    )(g, m, w, b[None, :])


def _ln(x, s, b):
    m = x.mean(-1, keepdims=True)
    v = ((x - m) ** 2).mean(-1, keepdims=True)
    return (x - m) / jnp.sqrt(v + 1e-5) * s + b


def _tf_body(y0_ref, l1s, l1b, wq, bq, wk, bk, wv, bv, wo, bo,
             l2s, l2b, wf1, bf1, wf2, bf2, wcls, bcls, out_ref, y):
    i = pl.program_id(0)

    @pl.when(i == 0)
    def _():
        y[...] = y0_ref[...]

    hd = D // HEADS
    colmask = lax.broadcasted_iota(jnp.int32, (SPAD, SPAD), 1) < (C + 1)

    t = _ln(y[...], l1s[0], l1b[0])
    q = jnp.dot(t, wq[0], preferred_element_type=F32) + bq[0]
    kk = jnp.dot(t, wk[0], preferred_element_type=F32) + bk[0]
    v = jnp.dot(t, wv[0], preferred_element_type=F32) + bv[0]
    ohs = []
    for h in range(HEADS):
        qh = q[:, h * hd:(h + 1) * hd]
        kh = kk[:, h * hd:(h + 1) * hd]
        vh = v[:, h * hd:(h + 1) * hd]
        sc = lax.dot_general(qh, kh, (((1,), (1,)), ((), ())),
                             preferred_element_type=F32) * (1.0 / 4.0)
        sc = jnp.where(colmask, sc, -1e30)
        sc = sc - sc.max(-1, keepdims=True)
        ex = jnp.exp(sc)
        att = ex / ex.sum(-1, keepdims=True)
        ohs.append(jnp.dot(att, vh, preferred_element_type=F32))
    o = jnp.concatenate(ohs, axis=1)
    ynew = y[...] + jnp.dot(o, wo[0], preferred_element_type=F32) + bo[0]
    t2 = _ln(ynew, l2s[0], l2b[0])
    f = jnp.dot(jax.nn.gelu(jnp.dot(t2, wf1[0], preferred_element_type=F32)
                            + bf1[0]),
                wf2[0], preferred_element_type=F32) + bf2[0]
    y[...] = ynew + f

    @pl.when(i == DEPTH - 1)
    def _():
        out_ref[...] = (jnp.dot(y[0:1, :], wcls[...],
                                preferred_element_type=F32) + bcls[...])


@jax.jit
def _transformer(y0, p):
    cube = lambda i: (i, 0, 0)
    fixed = lambda i: (0, 0)
    row3 = lambda n: pl.BlockSpec((1, 1, n), cube)
    r2 = lambda a: a[:, None, :]
    return pl.pallas_call(
        _tf_body,
        grid=(DEPTH,),
        in_specs=[
            pl.BlockSpec((SPAD, D), fixed),
            row3(D), row3(D),
            pl.BlockSpec((1, D, D), cube), row3(D),
            pl.BlockSpec((1, D, D), cube), row3(D),
            pl.BlockSpec((1, D, D), cube), row3(D),
            pl.BlockSpec((1, D, D), cube), row3(D),
            row3(D), row3(D),
            pl.BlockSpec((1, D, 4 * D), cube), row3(4 * D),
            pl.BlockSpec((1, 4 * D, D), cube), row3(D),
            pl.BlockSpec((D, NUM_CLASSES), fixed),
            pl.BlockSpec((1, NUM_CLASSES), fixed),
        ],
        out_specs=pl.BlockSpec((1, NUM_CLASSES), fixed),
        out_shape=jax.ShapeDtypeStruct((1, NUM_CLASSES), F32),
        scratch_shapes=[pltpu.VMEM((SPAD, D), F32)],
    )(y0, r2(p['ln1_s']), r2(p['ln1_b']), p['Wq'], r2(p['bq']),
      p['Wk'], r2(p['bk']), p['Wv'], r2(p['bv']), p['Wo'], r2(p['bo']),
      r2(p['ln2_s']), r2(p['ln2_b']),
      p['Wf1'], r2(p['bf1']), p['Wf2'], r2(p['bf2']),
      p['W_cls'], p['b_cls'][None, :])


def kernel(x, params, edge_index, cluster_indices, cluster_mask):
    p = params
    xp = jnp.pad(x, ((0, NPAD - N), (0, 0)))
    sl = jnp.arange(N, dtype=jnp.int32)
    npad_e = EPAD - E - N
    spread = N + (jnp.arange(npad_e, dtype=jnp.int32) % (NPAD - N))
    src = jnp.concatenate([edge_index[0].astype(jnp.int32), sl, spread])
    dst = jnp.concatenate([edge_index[1].astype(jnp.int32), sl, spread])
    src2 = src.reshape(32, 44, 128)
    dst2 = dst.reshape(32, 44, 128)
    iota_rows = jnp.arange(NROWS, dtype=jnp.int32).reshape(1, NROWS)

    z1, s1, d1 = _proj1(xp, p['W_in'], p['b_in'], p['W1'],
                        p['a_src1'], p['a_dst1'])
    pp1 = _edge_phase(z1, s1, d1, src2, dst2, iota_rows)
    z2, s2, d2 = _proj2(pp1, p['b1'], p['W2'], p['a_src2'], p['a_dst2'])
    pp2 = _edge_phase(z2, s2, d2, src2, dst2, iota_rows)
    h2 = _sum2(pp2, p['b2'])

    cit = jnp.pad(cluster_indices.T.astype(jnp.int32), ((0, 0), (0, 7)))
    idx = cit.reshape(32, 5, 64)
    mt = jnp.pad(cluster_mask.T, ((0, 0), (0, 7)))
    mbc = mt[:, :, None]

    g = _cluster_gather(h2, idx)
    ce = _emb(g.reshape(K, SPAD, D), mbc, p['W_emb'].reshape(K, D, D),
              p['b_emb'])
    y0 = jnp.concatenate([p['class_token'][None, :], ce[:C],
                          jnp.zeros((SPAD - C - 1, D), F32)], axis=0)
    return _transformer(y0, p)
